# Initial kernel scaffold; baseline (speedup 1.0000x reference)
import jax
import jax.numpy as jnp
from jax.experimental import pallas as pl


def kernel(n, d_i, d_j, edge_index, edge_weight, E_weight, W0, W1, W2):
    raise NotImplementedError("write your pallas kernel here")


# speedup vs baseline: 5.3557x; 5.3557x over previous
import functools

import jax
import jax.numpy as jnp
from jax import lax
from jax.experimental import pallas as pl
from jax.experimental.pallas import tpu as pltpu
from jax.experimental.pallas import tpu_sc as plsc

N_NUM = 8000
D_NUM = 2000
NT = N_NUM + D_NUM
E = 128
NE = 320000
B = 4096
REG = 1e-4

NC, NS, L = 2, 16, 16
NW = NC * NS
EPW = NE // NW
CH = 80
NCH = EPW // CH
NTP = 10240
RPT = NTP // NS
RLAST = NT - (NS - 1) * RPT

GN = 3 * B
GPW = GN // NW
GC = 128

_MESH = plsc.VectorSubcoreMesh(
    core_axis_name="c", subcore_axis_name="s", num_cores=NC, num_subcores=NS)

MM_BLK = 1000
MM_GRID = NT // MM_BLK


def _mm_first_body(x_ref, w_ref, s_ref):
    s_ref[...] = lax.dot_general(
        x_ref[...], w_ref[...], (((1,), (1,)), ((), ())),
        preferred_element_type=jnp.float32)


def _mm_first(x, w):
    return pl.pallas_call(
        _mm_first_body,
        grid=(MM_GRID,),
        in_specs=[pl.BlockSpec((MM_BLK, E), lambda i: (i, 0)),
                  pl.BlockSpec((E, E), lambda i: (0, 0))],
        out_specs=pl.BlockSpec((MM_BLK, E), lambda i: (i, 0)),
        out_shape=jax.ShapeDtypeStruct((NT, E), jnp.float32),
    )(x, w)


def _mm_sum_body(a_ref, b_ref, w_ref, x_ref, s_ref):
    x = a_ref[...] + b_ref[...]
    x_ref[...] = x
    s_ref[...] = lax.dot_general(
        x, w_ref[...], (((1,), (1,)), ((), ())),
        preferred_element_type=jnp.float32)


def _mm_sum(a, b, w):
    return pl.pallas_call(
        _mm_sum_body,
        grid=(MM_GRID,),
        in_specs=[pl.BlockSpec((MM_BLK, E), lambda i: (i, 0)),
                  pl.BlockSpec((MM_BLK, E), lambda i: (i, 0)),
                  pl.BlockSpec((E, E), lambda i: (0, 0))],
        out_specs=[pl.BlockSpec((MM_BLK, E), lambda i: (i, 0)),
                   pl.BlockSpec((MM_BLK, E), lambda i: (i, 0))],
        out_shape=[jax.ShapeDtypeStruct((NT, E), jnp.float32),
                   jax.ShapeDtypeStruct((NT, E), jnp.float32)],
    )(a, b, w)


def _add_body(a_ref, b_ref, o_ref):
    o_ref[...] = a_ref[...] + b_ref[...]


def _add2(a, b):
    return pl.pallas_call(
        _add_body,
        grid=(MM_GRID,),
        in_specs=[pl.BlockSpec((MM_BLK, E), lambda i: (i, 0)),
                  pl.BlockSpec((MM_BLK, E), lambda i: (i, 0))],
        out_specs=pl.BlockSpec((MM_BLK, E), lambda i: (i, 0)),
        out_shape=jax.ShapeDtypeStruct((NT, E), jnp.float32),
    )(a, b)


@functools.partial(
    pl.kernel,
    out_type=jax.ShapeDtypeStruct((NC * NT, E), jnp.float32),
    mesh=_MESH,
    scratch_types=[
        pltpu.VMEM((EPW,), jnp.int32),
        pltpu.VMEM((EPW,), jnp.int32),
        pltpu.VMEM((EPW,), jnp.float32),
        pltpu.VMEM((CH,), jnp.int32),
        pltpu.VMEM((CH,), jnp.int32),
        pltpu.VMEM((CH, E), jnp.float32),
        pltpu.VMEM_SHARED((NTP, E), jnp.float32),
        pltpu.SemaphoreType.DMA,
    ],
)
def _edge_kernel(sup_hbm, src_hbm, dst_hbm, w_hbm, z_hbm, out_hbm,
                 src_v, dst_v, w_v, srcc_v, dstc_v, rows_v, acc_sh, sem):
    cid = lax.axis_index("c")
    sid = lax.axis_index("s")
    wid = cid * NS + sid
    ebase = wid * EPW

    pltpu.sync_copy(z_hbm.at[pl.ds(sid * RPT, RPT)],
                    acc_sh.at[pl.ds(sid * RPT, RPT)])

    pltpu.sync_copy(src_hbm.at[pl.ds(ebase, EPW)], src_v)
    pltpu.sync_copy(dst_hbm.at[pl.ds(ebase, EPW)], dst_v)
    pltpu.sync_copy(w_hbm.at[pl.ds(ebase, EPW)], w_v)

    plsc.subcore_barrier()

    @pl.loop(0, NCH)
    def _chunk(ci):
        cb = ci * CH
        for g in range(CH // L):
            srcc_v[pl.ds(g * L, L)] = src_v[pl.ds(cb + g * L, L)]
            dstc_v[pl.ds(g * L, L)] = dst_v[pl.ds(cb + g * L, L)]
        pltpu.async_copy(sup_hbm.at[srcc_v], rows_v, sem).wait()
        for g in range(CH // L):
            wg = w_v[pl.ds(cb + g * L, L)]
            for e in range(L):
                row = g * L + e
                wb = lax.gather(
                    wg, jnp.full((L, 1), e, jnp.int32),
                    lax.GatherDimensionNumbers(
                        offset_dims=(), collapsed_slice_dims=(0,),
                        start_index_map=(0,)),
                    slice_sizes=(1,),
                    mode=lax.GatherScatterMode.PROMISE_IN_BOUNDS)
                for f in range(E // L):
                    rows_v[row, pl.ds(f * L, L)] = (
                        rows_v[row, pl.ds(f * L, L)] * wb)
        pltpu.sync_copy(rows_v, acc_sh.at[dstc_v], add=True)

    plsc.subcore_barrier()

    @pl.when(sid < NS - 1)
    def _():
        pltpu.sync_copy(acc_sh.at[pl.ds(sid * RPT, RPT)],
                        out_hbm.at[pl.ds(cid * NT + sid * RPT, RPT)])

    @pl.when(sid == NS - 1)
    def _():
        pltpu.sync_copy(
            acc_sh.at[pl.ds((NS - 1) * RPT, RLAST)],
            out_hbm.at[pl.ds(cid * NT + (NS - 1) * RPT, RLAST)])


@functools.partial(
    pl.kernel,
    out_type=[jax.ShapeDtypeStruct((GN, E), jnp.float32) for _ in range(4)],
    mesh=_MESH,
    scratch_types=[
        pltpu.VMEM((GPW,), jnp.int32),
        pltpu.VMEM((GC,), jnp.int32),
        pltpu.VMEM((GC, E), jnp.float32),
        pltpu.SemaphoreType.DMA,
    ],
)
def _gather_kernel(idx_hbm, t0, t1, t2, t3, o0, o1, o2, o3,
                   idx_v, idxc_v, rows_v, sem):
    cid = lax.axis_index("c")
    sid = lax.axis_index("s")
    wid = cid * NS + sid
    base = wid * GPW

    pltpu.sync_copy(idx_hbm.at[pl.ds(base, GPW)], idx_v)
    for g in range(GPW // L):
        gpos = jnp.full((L,), base + g * L, jnp.int32) + lax.iota(jnp.int32, L)
        v = idx_v[pl.ds(g * L, L)]
        off = jnp.where(gpos >= B,
                        jnp.full((L,), N_NUM, jnp.int32),
                        jnp.zeros((L,), jnp.int32))
        idx_v[pl.ds(g * L, L)] = v + off

    for tbl, out in ((t0, o0), (t1, o1), (t2, o2), (t3, o3)):
        for k in range(GPW // GC):
            for g in range(GC // L):
                idxc_v[pl.ds(g * L, L)] = idx_v[pl.ds(k * GC + g * L, L)]
            pltpu.async_copy(tbl.at[idxc_v], rows_v, sem).wait()
            pltpu.sync_copy(rows_v, out.at[pl.ds(base + k * GC, GC)])


LB = 1024
LSTEPS = B // LB
LBR = LB // E


def _loss_body(g0n, g0i, g0j, g1n, g1i, g1j, g2n, g2i, g2j, g3n, g3i, g3j,
               w0_ref, w1_ref, w2_ref, pi_ref, pj_ref, loss_ref, acc_ref):
    step = pl.program_id(0)
    pi = jnp.zeros((LB,), jnp.float32)
    pj = jnp.zeros((LB,), jnp.float32)
    sq = jnp.zeros((LB,), jnp.float32)
    for gn, gi, gj in ((g0n, g0i, g0j), (g1n, g1i, g1j),
                       (g2n, g2i, g2j), (g3n, g3i, g3j)):
        a = gn[...]
        bi = gi[...]
        bj = gj[...]
        pi = pi + jnp.sum(a * bi, axis=1)
        pj = pj + jnp.sum(a * bj, axis=1)
        sq = sq + jnp.sum(a * a + bi * bi + bj * bj, axis=1)
    pi_ref[...] = pi.reshape(LBR, E)
    pj_ref[...] = pj.reshape(LBR, E)

    z = pi - pj
    logsig = -(jnp.maximum(-z, 0.0) + jnp.log1p(jnp.exp(-jnp.abs(z))))

    @pl.when(step == 0)
    def _():
        acc_ref[0] = 0.0
        acc_ref[1] = 0.0

    acc_ref[0] = acc_ref[0] + jnp.sum(logsig)
    acc_ref[1] = acc_ref[1] + jnp.sum(sq)

    frob = (jnp.sqrt(jnp.sum(w0_ref[...] ** 2))
            + jnp.sqrt(jnp.sum(w1_ref[...] ** 2))
            + jnp.sqrt(jnp.sum(w2_ref[...] ** 2)))
    l2 = frob + acc_ref[1] / B
    loss = -acc_ref[0] / B + REG * l2
    loss_ref[...] = jnp.full((1, E), loss, jnp.float32)


def _loss(G0, G1, G2, G3, W0, W1, W2):
    g_specs = []
    g_args = []
    for G in (G0, G1, G2, G3):
        for p in range(3):
            g_specs.append(pl.BlockSpec(
                (LB, E), lambda i, p=p: (p * LSTEPS + i, 0)))
            g_args.append(G)
    w_specs = [pl.BlockSpec((E, E), lambda i: (0, 0)) for _ in range(3)]
    return pl.pallas_call(
        _loss_body,
        grid=(LSTEPS,),
        in_specs=g_specs + w_specs,
        out_specs=[pl.BlockSpec((LBR, E), lambda i: (i, 0)),
                   pl.BlockSpec((LBR, E), lambda i: (i, 0)),
                   pl.BlockSpec((1, E), lambda i: (0, 0))],
        out_shape=[jax.ShapeDtypeStruct((B // E, E), jnp.float32),
                   jax.ShapeDtypeStruct((B // E, E), jnp.float32),
                   jax.ShapeDtypeStruct((1, E), jnp.float32)],
        scratch_shapes=[pltpu.SMEM((2,), jnp.float32)],
    )(*g_args, W0, W1, W2)


def kernel(n, d_i, d_j, edge_index, edge_weight, E_weight, W0, W1, W2):
    src = edge_index[0].astype(jnp.int32)
    dst = edge_index[1].astype(jnp.int32)
    zeros_tbl = jnp.zeros((NTP, E), jnp.float32)

    s0 = _mm_first(E_weight, W0)
    h = _edge_kernel(s0, src, dst, edge_weight, zeros_tbl)
    x1, s1 = _mm_sum(h[:NT], h[NT:], W1)
    h = _edge_kernel(s1, src, dst, edge_weight, zeros_tbl)
    x2, s2 = _mm_sum(h[:NT], h[NT:], W2)
    h = _edge_kernel(s2, src, dst, edge_weight, zeros_tbl)
    x3 = _add2(h[:NT], h[NT:])

    idx_cat = jnp.concatenate(
        [n.astype(jnp.int32), d_i.astype(jnp.int32), d_j.astype(jnp.int32)])
    G0, G1, G2, G3 = _gather_kernel(idx_cat, E_weight, x1, x2, x3)

    pre_i, pre_j, loss_buf = _loss(G0, G1, G2, G3, W0, W1, W2)
    return pre_i.reshape(B), pre_j.reshape(B), loss_buf[0, :1]


# speedup vs baseline: 8.0714x; 1.5071x over previous
import functools

import jax
import jax.numpy as jnp
from jax import lax
from jax.experimental import pallas as pl
from jax.experimental.pallas import tpu as pltpu
from jax.experimental.pallas import tpu_sc as plsc

N_NUM = 8000
D_NUM = 2000
NT = N_NUM + D_NUM
E = 128
NE = 320000
B = 4096
REG = 1e-4

NC, NS, L = 2, 16, 16
NW = NC * NS
EPW = NE // NW
CH = 80
NCH = EPW // CH
ZS = 1000
NZT = NT // ZS

GN = 3 * B
GPW = GN // NW
GC = 128

_MESH = plsc.VectorSubcoreMesh(
    core_axis_name="c", subcore_axis_name="s", num_cores=NC, num_subcores=NS)

MM_BLK = 1000
MM_GRID = NT // MM_BLK


def _mm_first_body(x_ref, w_ref, s_ref):
    s_ref[...] = lax.dot_general(
        x_ref[...], w_ref[...], (((1,), (1,)), ((), ())),
        preferred_element_type=jnp.float32)


def _mm_first(x, w):
    return pl.pallas_call(
        _mm_first_body,
        grid=(MM_GRID,),
        in_specs=[pl.BlockSpec((MM_BLK, E), lambda i: (i, 0)),
                  pl.BlockSpec((E, E), lambda i: (0, 0))],
        out_specs=pl.BlockSpec((MM_BLK, E), lambda i: (i, 0)),
        out_shape=jax.ShapeDtypeStruct((NT, E), jnp.float32),
    )(x, w)


def _mm_sum_body(a_ref, b_ref, w_ref, x_ref, s_ref):
    x = a_ref[...] + b_ref[...]
    x_ref[...] = x
    s_ref[...] = lax.dot_general(
        x, w_ref[...], (((1,), (1,)), ((), ())),
        preferred_element_type=jnp.float32)


def _mm_sum(a, b, w):
    return pl.pallas_call(
        _mm_sum_body,
        grid=(MM_GRID,),
        in_specs=[pl.BlockSpec((MM_BLK, E), lambda i: (i, 0)),
                  pl.BlockSpec((MM_BLK, E), lambda i: (i, 0)),
                  pl.BlockSpec((E, E), lambda i: (0, 0))],
        out_specs=[pl.BlockSpec((MM_BLK, E), lambda i: (i, 0)),
                   pl.BlockSpec((MM_BLK, E), lambda i: (i, 0))],
        out_shape=[jax.ShapeDtypeStruct((NT, E), jnp.float32),
                   jax.ShapeDtypeStruct((NT, E), jnp.float32)],
    )(a, b, w)


def _add_body(a_ref, b_ref, o_ref):
    o_ref[...] = a_ref[...] + b_ref[...]


def _add2(a, b):
    return pl.pallas_call(
        _add_body,
        grid=(MM_GRID,),
        in_specs=[pl.BlockSpec((MM_BLK, E), lambda i: (i, 0)),
                  pl.BlockSpec((MM_BLK, E), lambda i: (i, 0))],
        out_specs=pl.BlockSpec((MM_BLK, E), lambda i: (i, 0)),
        out_shape=jax.ShapeDtypeStruct((NT, E), jnp.float32),
    )(a, b)


@functools.partial(
    pl.kernel,
    out_type=jax.ShapeDtypeStruct((NC * NT, E), jnp.float32),
    mesh=_MESH,
    scratch_types=[
        pltpu.VMEM((EPW,), jnp.int32),
        pltpu.VMEM((EPW,), jnp.float32),
        pltpu.VMEM((CH,), jnp.int32),
        pltpu.VMEM((CH,), jnp.int32),
        pltpu.VMEM((CH, E), jnp.float32),
        pltpu.VMEM((CH, E), jnp.float32),
        pltpu.VMEM_SHARED((NT, E), jnp.float32),
        pltpu.SemaphoreType.DMA,
        pltpu.SemaphoreType.DMA,
        pltpu.SemaphoreType.DMA,
        pltpu.SemaphoreType.DMA,
        pltpu.SemaphoreType.DMA,
        pltpu.SemaphoreType.DMA,
    ],
)
def _edge_kernel(sup_hbm, src_hbm, dst_hbm, w_hbm, z_hbm, out_hbm,
                 src_v, w_v, dstc0, dstc1, rows0, rows1, acc_sh,
                 sg0, sg1, ss0, ss1, sd0, sd1):
    cid = lax.axis_index("c")
    sid = lax.axis_index("s")
    wid = cid * NS + sid
    ebase = wid * EPW

    @pl.when(sid < NZT)
    def _():
        pltpu.sync_copy(z_hbm.at[pl.ds(sid * ZS, ZS)],
                        acc_sh.at[pl.ds(sid * ZS, ZS)])

    pltpu.sync_copy(src_hbm.at[pl.ds(ebase, EPW)], src_v)
    pltpu.sync_copy(w_hbm.at[pl.ds(ebase, EPW)], w_v)

    plsc.subcore_barrier()

    def g_start(ci, rows, sem):
        pltpu.async_copy(sup_hbm.at[src_v.at[pl.ds(ci * CH, CH)]], rows, sem)

    def g_wait(ci, rows, sem):
        pltpu.make_async_copy(
            sup_hbm.at[src_v.at[pl.ds(ci * CH, CH)]], rows, sem).wait()

    def d_start(ci, dstc, sem):
        pltpu.async_copy(dst_hbm.at[pl.ds(ebase + ci * CH, CH)], dstc, sem)

    def d_wait(ci, dstc, sem):
        pltpu.make_async_copy(
            dst_hbm.at[pl.ds(ebase + ci * CH, CH)], dstc, sem).wait()

    def s_start(rows, dstc, sem):
        pltpu.async_copy(rows, acc_sh.at[dstc], sem, add=True)

    def s_wait(rows, dstc, sem):
        pltpu.make_async_copy(rows, acc_sh.at[dstc], sem).wait()

    def scale(ci, rows):
        cb = ci * CH

        @pl.loop(0, CH // L)
        def _grp(g):
            wg = w_v[pl.ds(cb + g * L, L)]
            for e in range(L):
                row = g * L + e
                wb = lax.gather(
                    wg, jnp.full((L, 1), e, jnp.int32),
                    lax.GatherDimensionNumbers(
                        offset_dims=(), collapsed_slice_dims=(0,),
                        start_index_map=(0,)),
                    slice_sizes=(1,),
                    mode=lax.GatherScatterMode.PROMISE_IN_BOUNDS)
                for f in range(E // L):
                    rows[row, pl.ds(f * L, L)] = (
                        rows[row, pl.ds(f * L, L)] * wb)

    d_start(0, dstc0, sd0)
    g_start(0, rows0, sg0)
    d_start(1, dstc1, sd1)
    g_wait(0, rows0, sg0)
    g_start(1, rows1, sg1)
    scale(0, rows0)
    d_wait(0, dstc0, sd0)
    s_start(rows0, dstc0, ss0)

    @pl.loop(0, (NCH - 3) // 2)
    def _pair(i):
        ci = 2 * i + 1
        g_wait(ci, rows1, sg1)
        s_wait(rows0, dstc0, ss0)
        d_start(ci + 1, dstc0, sd0)
        g_start(ci + 1, rows0, sg0)
        scale(ci, rows1)
        d_wait(ci, dstc1, sd1)
        s_start(rows1, dstc1, ss1)
        g_wait(ci + 1, rows0, sg0)
        s_wait(rows1, dstc1, ss1)
        d_start(ci + 2, dstc1, sd1)
        g_start(ci + 2, rows1, sg1)
        scale(ci + 1, rows0)
        d_wait(ci + 1, dstc0, sd0)
        s_start(rows0, dstc0, ss0)

    g_wait(NCH - 2, rows1, sg1)
    s_wait(rows0, dstc0, ss0)
    d_start(NCH - 1, dstc0, sd0)
    g_start(NCH - 1, rows0, sg0)
    scale(NCH - 2, rows1)
    d_wait(NCH - 2, dstc1, sd1)
    s_start(rows1, dstc1, ss1)

    g_wait(NCH - 1, rows0, sg0)
    scale(NCH - 1, rows0)
    d_wait(NCH - 1, dstc0, sd0)
    s_start(rows0, dstc0, ss0)

    s_wait(rows1, dstc1, ss1)
    s_wait(rows0, dstc0, ss0)

    plsc.subcore_barrier()

    @pl.when(sid < NZT)
    def _():
        pltpu.sync_copy(acc_sh.at[pl.ds(sid * ZS, ZS)],
                        out_hbm.at[pl.ds(cid * NT + sid * ZS, ZS)])


@functools.partial(
    pl.kernel,
    out_type=[jax.ShapeDtypeStruct((GN, E), jnp.float32) for _ in range(4)],
    mesh=_MESH,
    scratch_types=[
        pltpu.VMEM((GPW,), jnp.int32),
        pltpu.VMEM((GC,), jnp.int32),
        pltpu.VMEM((GC, E), jnp.float32),
        pltpu.SemaphoreType.DMA,
    ],
)
def _gather_kernel(idx_hbm, t0, t1, t2, t3, o0, o1, o2, o3,
                   idx_v, idxc_v, rows_v, sem):
    cid = lax.axis_index("c")
    sid = lax.axis_index("s")
    wid = cid * NS + sid
    base = wid * GPW

    pltpu.sync_copy(idx_hbm.at[pl.ds(base, GPW)], idx_v)
    for g in range(GPW // L):
        gpos = jnp.full((L,), base + g * L, jnp.int32) + lax.iota(jnp.int32, L)
        v = idx_v[pl.ds(g * L, L)]
        off = jnp.where(gpos >= B,
                        jnp.full((L,), N_NUM, jnp.int32),
                        jnp.zeros((L,), jnp.int32))
        idx_v[pl.ds(g * L, L)] = v + off

    for tbl, out in ((t0, o0), (t1, o1), (t2, o2), (t3, o3)):
        for k in range(GPW // GC):
            for g in range(GC // L):
                idxc_v[pl.ds(g * L, L)] = idx_v[pl.ds(k * GC + g * L, L)]
            pltpu.async_copy(tbl.at[idxc_v], rows_v, sem).wait()
            pltpu.sync_copy(rows_v, out.at[pl.ds(base + k * GC, GC)])


LB = 1024
LSTEPS = B // LB
LBR = LB // E


def _loss_body(g0n, g0i, g0j, g1n, g1i, g1j, g2n, g2i, g2j, g3n, g3i, g3j,
               w0_ref, w1_ref, w2_ref, pi_ref, pj_ref, loss_ref, acc_ref):
    step = pl.program_id(0)
    pi = jnp.zeros((LB,), jnp.float32)
    pj = jnp.zeros((LB,), jnp.float32)
    sq = jnp.zeros((LB,), jnp.float32)
    for gn, gi, gj in ((g0n, g0i, g0j), (g1n, g1i, g1j),
                       (g2n, g2i, g2j), (g3n, g3i, g3j)):
        a = gn[...]
        bi = gi[...]
        bj = gj[...]
        pi = pi + jnp.sum(a * bi, axis=1)
        pj = pj + jnp.sum(a * bj, axis=1)
        sq = sq + jnp.sum(a * a + bi * bi + bj * bj, axis=1)
    pi_ref[...] = pi.reshape(LBR, E)
    pj_ref[...] = pj.reshape(LBR, E)

    z = pi - pj
    logsig = -(jnp.maximum(-z, 0.0) + jnp.log1p(jnp.exp(-jnp.abs(z))))

    @pl.when(step == 0)
    def _():
        acc_ref[0] = 0.0
        acc_ref[1] = 0.0

    acc_ref[0] = acc_ref[0] + jnp.sum(logsig)
    acc_ref[1] = acc_ref[1] + jnp.sum(sq)

    frob = (jnp.sqrt(jnp.sum(w0_ref[...] ** 2))
            + jnp.sqrt(jnp.sum(w1_ref[...] ** 2))
            + jnp.sqrt(jnp.sum(w2_ref[...] ** 2)))
    l2 = frob + acc_ref[1] / B
    loss = -acc_ref[0] / B + REG * l2
    loss_ref[...] = jnp.full((1, E), loss, jnp.float32)


def _loss(G0, G1, G2, G3, W0, W1, W2):
    g_specs = []
    g_args = []
    for G in (G0, G1, G2, G3):
        for p in range(3):
            g_specs.append(pl.BlockSpec(
                (LB, E), lambda i, p=p: (p * LSTEPS + i, 0)))
            g_args.append(G)
    w_specs = [pl.BlockSpec((E, E), lambda i: (0, 0)) for _ in range(3)]
    return pl.pallas_call(
        _loss_body,
        grid=(LSTEPS,),
        in_specs=g_specs + w_specs,
        out_specs=[pl.BlockSpec((LBR, E), lambda i: (i, 0)),
                   pl.BlockSpec((LBR, E), lambda i: (i, 0)),
                   pl.BlockSpec((1, E), lambda i: (0, 0))],
        out_shape=[jax.ShapeDtypeStruct((B // E, E), jnp.float32),
                   jax.ShapeDtypeStruct((B // E, E), jnp.float32),
                   jax.ShapeDtypeStruct((1, E), jnp.float32)],
        scratch_shapes=[pltpu.SMEM((2,), jnp.float32)],
    )(*g_args, W0, W1, W2)


def kernel(n, d_i, d_j, edge_index, edge_weight, E_weight, W0, W1, W2):
    src = edge_index[0].astype(jnp.int32)
    dst = edge_index[1].astype(jnp.int32)
    zeros_tbl = jnp.zeros((NT, E), jnp.float32)

    s0 = _mm_first(E_weight, W0)
    h = _edge_kernel(s0, src, dst, edge_weight, zeros_tbl)
    x1, s1 = _mm_sum(h[:NT], h[NT:], W1)
    h = _edge_kernel(s1, src, dst, edge_weight, zeros_tbl)
    x2, s2 = _mm_sum(h[:NT], h[NT:], W2)
    h = _edge_kernel(s2, src, dst, edge_weight, zeros_tbl)
    x3 = _add2(h[:NT], h[NT:])

    idx_cat = jnp.concatenate(
        [n.astype(jnp.int32), d_i.astype(jnp.int32), d_j.astype(jnp.int32)])
    G0, G1, G2, G3 = _gather_kernel(idx_cat, E_weight, x1, x2, x3)

    pre_i, pre_j, loss_buf = _loss(G0, G1, G2, G3, W0, W1, W2)
    return pre_i.reshape(B), pre_j.reshape(B), loss_buf[0, :1]


# speedup vs baseline: 9.0009x; 1.1152x over previous
import functools

import jax
import jax.numpy as jnp
from jax import lax
from jax.experimental import pallas as pl
from jax.experimental.pallas import tpu as pltpu
from jax.experimental.pallas import tpu_sc as plsc

N_NUM = 8000
D_NUM = 2000
NT = N_NUM + D_NUM
E = 128
NE = 320000
B = 4096
REG = 1e-4

NC, NS, L = 2, 16, 16
NW = NC * NS
EPW = NE // NW
CH = 128
NCH = EPW // CH
CT = EPW - NCH * CH
ZS = 1000
NZT = NT // ZS

GN = 3 * B
GPW = GN // NW
GC = 128

_MESH = plsc.VectorSubcoreMesh(
    core_axis_name="c", subcore_axis_name="s", num_cores=NC, num_subcores=NS)

MM_BLK = 1000
MM_GRID = NT // MM_BLK


def _mm_first_body(x_ref, w_ref, s_ref):
    s_ref[...] = lax.dot_general(
        x_ref[...], w_ref[...], (((1,), (1,)), ((), ())),
        preferred_element_type=jnp.float32)


def _mm_first(x, w):
    return pl.pallas_call(
        _mm_first_body,
        grid=(MM_GRID,),
        in_specs=[pl.BlockSpec((MM_BLK, E), lambda i: (i, 0)),
                  pl.BlockSpec((E, E), lambda i: (0, 0))],
        out_specs=pl.BlockSpec((MM_BLK, E), lambda i: (i, 0)),
        out_shape=jax.ShapeDtypeStruct((NT, E), jnp.float32),
    )(x, w)


def _mm_sum_body(a_ref, b_ref, w_ref, x_ref, s_ref):
    x = a_ref[...] + b_ref[...]
    x_ref[...] = x
    s_ref[...] = lax.dot_general(
        x, w_ref[...], (((1,), (1,)), ((), ())),
        preferred_element_type=jnp.float32)


def _mm_sum(a, b, w):
    return pl.pallas_call(
        _mm_sum_body,
        grid=(MM_GRID,),
        in_specs=[pl.BlockSpec((MM_BLK, E), lambda i: (i, 0)),
                  pl.BlockSpec((MM_BLK, E), lambda i: (i, 0)),
                  pl.BlockSpec((E, E), lambda i: (0, 0))],
        out_specs=[pl.BlockSpec((MM_BLK, E), lambda i: (i, 0)),
                   pl.BlockSpec((MM_BLK, E), lambda i: (i, 0))],
        out_shape=[jax.ShapeDtypeStruct((NT, E), jnp.float32),
                   jax.ShapeDtypeStruct((NT, E), jnp.float32)],
    )(a, b, w)


def _add_body(a_ref, b_ref, o_ref):
    o_ref[...] = a_ref[...] + b_ref[...]


def _add2(a, b):
    return pl.pallas_call(
        _add_body,
        grid=(MM_GRID,),
        in_specs=[pl.BlockSpec((MM_BLK, E), lambda i: (i, 0)),
                  pl.BlockSpec((MM_BLK, E), lambda i: (i, 0))],
        out_specs=pl.BlockSpec((MM_BLK, E), lambda i: (i, 0)),
        out_shape=jax.ShapeDtypeStruct((NT, E), jnp.float32),
    )(a, b)


@functools.partial(
    pl.kernel,
    out_type=jax.ShapeDtypeStruct((NC * NT, E), jnp.float32),
    mesh=_MESH,
    scratch_types=[
        pltpu.VMEM((EPW,), jnp.float32),
        pltpu.VMEM((CH,), jnp.int32),
        pltpu.VMEM((CH,), jnp.int32),
        pltpu.VMEM((CH,), jnp.int32),
        pltpu.VMEM((CH,), jnp.int32),
        pltpu.VMEM((CT,), jnp.int32),
        pltpu.VMEM((CT,), jnp.int32),
        pltpu.VMEM((CH, E), jnp.float32),
        pltpu.VMEM((CH, E), jnp.float32),
        pltpu.VMEM_SHARED((NT, E), jnp.float32),
        pltpu.SemaphoreType.DMA,
        pltpu.SemaphoreType.DMA,
        pltpu.SemaphoreType.DMA,
        pltpu.SemaphoreType.DMA,
        pltpu.SemaphoreType.DMA,
        pltpu.SemaphoreType.DMA,
        pltpu.SemaphoreType.DMA,
        pltpu.SemaphoreType.DMA,
    ],
)
def _edge_kernel(sup_hbm, src_hbm, dst_hbm, w_hbm, z_hbm, out_hbm,
                 w_v, srcc0, srcc1, dstc0, dstc1, srcct, dstct, rows0, rows1,
                 acc_sh, sg0, sg1, ss0, ss1, sd0, sd1, ssc0, ssc1):
    cid = lax.axis_index("c")
    sid = lax.axis_index("s")
    wid = cid * NS + sid
    ebase = wid * EPW

    @pl.when(sid < NZT)
    def _():
        pltpu.sync_copy(z_hbm.at[pl.ds(sid * ZS, ZS)],
                        acc_sh.at[pl.ds(sid * ZS, ZS)])

    pltpu.sync_copy(w_hbm.at[pl.ds(ebase, EPW)], w_v)

    plsc.subcore_barrier()

    def lanebcast(wg, e):
        return lax.gather(
            wg, jnp.full((L, 1), e, jnp.int32),
            lax.GatherDimensionNumbers(
                offset_dims=(), collapsed_slice_dims=(0,),
                start_index_map=(0,)),
            slice_sizes=(1,),
            mode=lax.GatherScatterMode.PROMISE_IN_BOUNDS)

    pltpu.sync_copy(src_hbm.at[pl.ds(ebase + NCH * CH, CT)], srcct)
    pltpu.sync_copy(dst_hbm.at[pl.ds(ebase + NCH * CH, CT)], dstct)
    pltpu.async_copy(sup_hbm.at[srcct], rows0.at[pl.ds(0, CT)], sg0).wait()
    wg = w_v[pl.ds(NCH * CH, L)]
    for e in range(CT):
        wb = lanebcast(wg, e)
        for f in range(E // L):
            rows0[e, pl.ds(f * L, L)] = rows0[e, pl.ds(f * L, L)] * wb
    pltpu.sync_copy(rows0.at[pl.ds(0, CT)], acc_sh.at[dstct], add=True)

    def sc_start(ci, srcc, sem):
        pltpu.async_copy(src_hbm.at[pl.ds(ebase + ci * CH, CH)], srcc, sem)

    def sc_wait(ci, srcc, sem):
        pltpu.make_async_copy(
            src_hbm.at[pl.ds(ebase + ci * CH, CH)], srcc, sem).wait()

    def g_start(srcc, rows, sem):
        pltpu.async_copy(sup_hbm.at[srcc], rows, sem)

    def g_wait(srcc, rows, sem):
        pltpu.make_async_copy(sup_hbm.at[srcc], rows, sem).wait()

    def d_start(ci, dstc, sem):
        pltpu.async_copy(dst_hbm.at[pl.ds(ebase + ci * CH, CH)], dstc, sem)

    def d_wait(ci, dstc, sem):
        pltpu.make_async_copy(
            dst_hbm.at[pl.ds(ebase + ci * CH, CH)], dstc, sem).wait()

    def s_start(rows, dstc, sem):
        pltpu.async_copy(rows, acc_sh.at[dstc], sem, add=True)

    def s_wait(rows, dstc, sem):
        pltpu.make_async_copy(rows, acc_sh.at[dstc], sem).wait()

    def scale(ci, rows):
        cb = ci * CH

        @pl.loop(0, CH // L)
        def _grp(g):
            wg = w_v[pl.ds(cb + g * L, L)]
            for e in range(L):
                row = g * L + e
                wb = lanebcast(wg, e)
                for f in range(E // L):
                    rows[row, pl.ds(f * L, L)] = (
                        rows[row, pl.ds(f * L, L)] * wb)

    sc_start(0, srcc0, ssc0)
    d_start(0, dstc0, sd0)
    sc_start(1, srcc1, ssc1)
    d_start(1, dstc1, sd1)
    sc_wait(0, srcc0, ssc0)
    g_start(srcc0, rows0, sg0)
    g_wait(srcc0, rows0, sg0)
    sc_start(2, srcc0, ssc0)
    sc_wait(1, srcc1, ssc1)
    g_start(srcc1, rows1, sg1)
    scale(0, rows0)
    d_wait(0, dstc0, sd0)
    s_start(rows0, dstc0, ss0)

    NPAIR = (NCH - 2) // 2

    @pl.loop(0, NPAIR)
    def _pair(i):
        ci = 2 * i + 1
        g_wait(srcc1, rows1, sg1)
        sc_start(ci + 2, srcc1, ssc1)
        s_wait(rows0, dstc0, ss0)
        d_start(ci + 1, dstc0, sd0)
        sc_wait(ci + 1, srcc0, ssc0)
        g_start(srcc0, rows0, sg0)
        scale(ci, rows1)
        d_wait(ci, dstc1, sd1)
        s_start(rows1, dstc1, ss1)
        g_wait(srcc0, rows0, sg0)

        @pl.when(i < NPAIR - 1)
        def _():
            sc_start(ci + 3, srcc0, ssc0)

        s_wait(rows1, dstc1, ss1)
        d_start(ci + 2, dstc1, sd1)
        sc_wait(ci + 2, srcc1, ssc1)
        g_start(srcc1, rows1, sg1)
        scale(ci + 1, rows0)
        d_wait(ci + 1, dstc0, sd0)
        s_start(rows0, dstc0, ss0)

    g_wait(srcc1, rows1, sg1)
    s_wait(rows0, dstc0, ss0)
    scale(NCH - 1, rows1)
    d_wait(NCH - 1, dstc1, sd1)
    s_start(rows1, dstc1, ss1)
    s_wait(rows1, dstc1, ss1)

    plsc.subcore_barrier()

    @pl.when(sid < NZT)
    def _():
        pltpu.sync_copy(acc_sh.at[pl.ds(sid * ZS, ZS)],
                        out_hbm.at[pl.ds(cid * NT + sid * ZS, ZS)])


@functools.partial(
    pl.kernel,
    out_type=[jax.ShapeDtypeStruct((GN, E), jnp.float32) for _ in range(4)],
    mesh=_MESH,
    scratch_types=[
        pltpu.VMEM((GPW,), jnp.int32),
        pltpu.VMEM((GC,), jnp.int32),
        pltpu.VMEM((GC, E), jnp.float32),
        pltpu.SemaphoreType.DMA,
    ],
)
def _gather_kernel(idx_hbm, t0, t1, t2, t3, o0, o1, o2, o3,
                   idx_v, idxc_v, rows_v, sem):
    cid = lax.axis_index("c")
    sid = lax.axis_index("s")
    wid = cid * NS + sid
    base = wid * GPW

    pltpu.sync_copy(idx_hbm.at[pl.ds(base, GPW)], idx_v)
    for g in range(GPW // L):
        gpos = jnp.full((L,), base + g * L, jnp.int32) + lax.iota(jnp.int32, L)
        v = idx_v[pl.ds(g * L, L)]
        off = jnp.where(gpos >= B,
                        jnp.full((L,), N_NUM, jnp.int32),
                        jnp.zeros((L,), jnp.int32))
        idx_v[pl.ds(g * L, L)] = v + off

    for tbl, out in ((t0, o0), (t1, o1), (t2, o2), (t3, o3)):
        for k in range(GPW // GC):
            for g in range(GC // L):
                idxc_v[pl.ds(g * L, L)] = idx_v[pl.ds(k * GC + g * L, L)]
            pltpu.async_copy(tbl.at[idxc_v], rows_v, sem).wait()
            pltpu.sync_copy(rows_v, out.at[pl.ds(base + k * GC, GC)])


LB = 1024
LSTEPS = B // LB
LBR = LB // E


def _loss_body(g0n, g0i, g0j, g1n, g1i, g1j, g2n, g2i, g2j, g3n, g3i, g3j,
               w0_ref, w1_ref, w2_ref, pi_ref, pj_ref, loss_ref, acc_ref):
    step = pl.program_id(0)
    pi = jnp.zeros((LB,), jnp.float32)
    pj = jnp.zeros((LB,), jnp.float32)
    sq = jnp.zeros((LB,), jnp.float32)
    for gn, gi, gj in ((g0n, g0i, g0j), (g1n, g1i, g1j),
                       (g2n, g2i, g2j), (g3n, g3i, g3j)):
        a = gn[...]
        bi = gi[...]
        bj = gj[...]
        pi = pi + jnp.sum(a * bi, axis=1)
        pj = pj + jnp.sum(a * bj, axis=1)
        sq = sq + jnp.sum(a * a + bi * bi + bj * bj, axis=1)
    pi_ref[...] = pi.reshape(LBR, E)
    pj_ref[...] = pj.reshape(LBR, E)

    z = pi - pj
    logsig = -(jnp.maximum(-z, 0.0) + jnp.log1p(jnp.exp(-jnp.abs(z))))

    @pl.when(step == 0)
    def _():
        acc_ref[0] = 0.0
        acc_ref[1] = 0.0

    acc_ref[0] = acc_ref[0] + jnp.sum(logsig)
    acc_ref[1] = acc_ref[1] + jnp.sum(sq)

    frob = (jnp.sqrt(jnp.sum(w0_ref[...] ** 2))
            + jnp.sqrt(jnp.sum(w1_ref[...] ** 2))
            + jnp.sqrt(jnp.sum(w2_ref[...] ** 2)))
    l2 = frob + acc_ref[1] / B
    loss = -acc_ref[0] / B + REG * l2
    loss_ref[...] = jnp.full((1, E), loss, jnp.float32)


def _loss(G0, G1, G2, G3, W0, W1, W2):
    g_specs = []
    g_args = []
    for G in (G0, G1, G2, G3):
        for p in range(3):
            g_specs.append(pl.BlockSpec(
                (LB, E), lambda i, p=p: (p * LSTEPS + i, 0)))
            g_args.append(G)
    w_specs = [pl.BlockSpec((E, E), lambda i: (0, 0)) for _ in range(3)]
    return pl.pallas_call(
        _loss_body,
        grid=(LSTEPS,),
        in_specs=g_specs + w_specs,
        out_specs=[pl.BlockSpec((LBR, E), lambda i: (i, 0)),
                   pl.BlockSpec((LBR, E), lambda i: (i, 0)),
                   pl.BlockSpec((1, E), lambda i: (0, 0))],
        out_shape=[jax.ShapeDtypeStruct((B // E, E), jnp.float32),
                   jax.ShapeDtypeStruct((B // E, E), jnp.float32),
                   jax.ShapeDtypeStruct((1, E), jnp.float32)],
        scratch_shapes=[pltpu.SMEM((2,), jnp.float32)],
    )(*g_args, W0, W1, W2)


def kernel(n, d_i, d_j, edge_index, edge_weight, E_weight, W0, W1, W2):
    src = edge_index[0].astype(jnp.int32)
    dst = edge_index[1].astype(jnp.int32)
    zeros_tbl = jnp.zeros((NT, E), jnp.float32)

    s0 = _mm_first(E_weight, W0)
    h = _edge_kernel(s0, src, dst, edge_weight, zeros_tbl)
    x1, s1 = _mm_sum(h[:NT], h[NT:], W1)
    h = _edge_kernel(s1, src, dst, edge_weight, zeros_tbl)
    x2, s2 = _mm_sum(h[:NT], h[NT:], W2)
    h = _edge_kernel(s2, src, dst, edge_weight, zeros_tbl)
    x3 = _add2(h[:NT], h[NT:])

    idx_cat = jnp.concatenate(
        [n.astype(jnp.int32), d_i.astype(jnp.int32), d_j.astype(jnp.int32)])
    G0, G1, G2, G3 = _gather_kernel(idx_cat, E_weight, x1, x2, x3)

    pre_i, pre_j, loss_buf = _loss(G0, G1, G2, G3, W0, W1, W2)
    return pre_i.reshape(B), pre_j.reshape(B), loss_buf[0, :1]


# speedup vs baseline: 9.1344x; 1.0148x over previous
import functools

import jax
import jax.numpy as jnp
from jax import lax
from jax.experimental import pallas as pl
from jax.experimental.pallas import tpu as pltpu
from jax.experimental.pallas import tpu_sc as plsc

N_NUM = 8000
D_NUM = 2000
NT = N_NUM + D_NUM
E = 128
NE = 320000
B = 4096
REG = 1e-4

NC, NS, L = 2, 16, 16
NW = NC * NS
EPW = NE // NW
CH = 192
NCH = EPW // CH
CT = EPW - NCH * CH
ZS = 1000
NZT = NT // ZS

GN = 3 * B
GPW = GN // NW
GC = 128

_MESH = plsc.VectorSubcoreMesh(
    core_axis_name="c", subcore_axis_name="s", num_cores=NC, num_subcores=NS)

MM_BLK = 1000
MM_GRID = NT // MM_BLK


def _mm_first_body(x_ref, w_ref, s_ref):
    s_ref[...] = lax.dot_general(
        x_ref[...], w_ref[...], (((1,), (1,)), ((), ())),
        preferred_element_type=jnp.float32)


def _mm_first(x, w):
    return pl.pallas_call(
        _mm_first_body,
        grid=(MM_GRID,),
        in_specs=[pl.BlockSpec((MM_BLK, E), lambda i: (i, 0)),
                  pl.BlockSpec((E, E), lambda i: (0, 0))],
        out_specs=pl.BlockSpec((MM_BLK, E), lambda i: (i, 0)),
        out_shape=jax.ShapeDtypeStruct((NT, E), jnp.float32),
    )(x, w)


def _mm_sum_body(a_ref, b_ref, w_ref, x_ref, s_ref):
    x = a_ref[...] + b_ref[...]
    x_ref[...] = x
    s_ref[...] = lax.dot_general(
        x, w_ref[...], (((1,), (1,)), ((), ())),
        preferred_element_type=jnp.float32)


def _mm_sum(a, b, w):
    return pl.pallas_call(
        _mm_sum_body,
        grid=(MM_GRID,),
        in_specs=[pl.BlockSpec((MM_BLK, E), lambda i: (i, 0)),
                  pl.BlockSpec((MM_BLK, E), lambda i: (i, 0)),
                  pl.BlockSpec((E, E), lambda i: (0, 0))],
        out_specs=[pl.BlockSpec((MM_BLK, E), lambda i: (i, 0)),
                   pl.BlockSpec((MM_BLK, E), lambda i: (i, 0))],
        out_shape=[jax.ShapeDtypeStruct((NT, E), jnp.float32),
                   jax.ShapeDtypeStruct((NT, E), jnp.float32)],
    )(a, b, w)


def _add_body(a_ref, b_ref, o_ref):
    o_ref[...] = a_ref[...] + b_ref[...]


def _add2(a, b):
    return pl.pallas_call(
        _add_body,
        grid=(MM_GRID,),
        in_specs=[pl.BlockSpec((MM_BLK, E), lambda i: (i, 0)),
                  pl.BlockSpec((MM_BLK, E), lambda i: (i, 0))],
        out_specs=pl.BlockSpec((MM_BLK, E), lambda i: (i, 0)),
        out_shape=jax.ShapeDtypeStruct((NT, E), jnp.float32),
    )(a, b)


@functools.partial(
    pl.kernel,
    out_type=jax.ShapeDtypeStruct((NC * NT, E), jnp.float32),
    mesh=_MESH,
    scratch_types=[
        pltpu.VMEM((CH,), jnp.int32),
        pltpu.VMEM((CH,), jnp.int32),
        pltpu.VMEM((CH,), jnp.int32),
        pltpu.VMEM((CH,), jnp.int32),
        pltpu.VMEM((CH,), jnp.float32),
        pltpu.VMEM((CH,), jnp.float32),
        pltpu.VMEM((CT,), jnp.int32),
        pltpu.VMEM((CT,), jnp.int32),
        pltpu.VMEM((CT,), jnp.float32),
        pltpu.VMEM((CH, E), jnp.float32),
        pltpu.VMEM((CH, E), jnp.float32),
        pltpu.VMEM_SHARED((NT, E), jnp.float32),
        pltpu.SemaphoreType.DMA,
        pltpu.SemaphoreType.DMA,
        pltpu.SemaphoreType.DMA,
        pltpu.SemaphoreType.DMA,
        pltpu.SemaphoreType.DMA,
        pltpu.SemaphoreType.DMA,
        pltpu.SemaphoreType.DMA,
        pltpu.SemaphoreType.DMA,
        pltpu.SemaphoreType.DMA,
        pltpu.SemaphoreType.DMA,
    ],
)
def _edge_kernel(sup_hbm, src_hbm, dst_hbm, w_hbm, z_hbm, out_hbm,
                 srcc0, srcc1, dstc0, dstc1, wc0, wc1, srcct, dstct, wct,
                 rows0, rows1,
                 acc_sh, sg0, sg1, ss0, ss1, sd0, sd1, ssc0, ssc1, sw0, sw1):
    cid = lax.axis_index("c")
    sid = lax.axis_index("s")
    wid = cid * NS + sid
    ebase = wid * EPW

    @pl.when(sid < NZT)
    def _():
        pltpu.sync_copy(z_hbm.at[pl.ds(sid * ZS, ZS)],
                        acc_sh.at[pl.ds(sid * ZS, ZS)])

    plsc.subcore_barrier()

    def lanebcast(wg, e):
        return lax.gather(
            wg, jnp.full((L, 1), e, jnp.int32),
            lax.GatherDimensionNumbers(
                offset_dims=(), collapsed_slice_dims=(0,),
                start_index_map=(0,)),
            slice_sizes=(1,),
            mode=lax.GatherScatterMode.PROMISE_IN_BOUNDS)

    pltpu.sync_copy(src_hbm.at[pl.ds(ebase + NCH * CH, CT)], srcct)
    pltpu.sync_copy(dst_hbm.at[pl.ds(ebase + NCH * CH, CT)], dstct)
    pltpu.sync_copy(w_hbm.at[pl.ds(ebase + NCH * CH, CT)], wct)
    pltpu.async_copy(sup_hbm.at[srcct], rows0.at[pl.ds(0, CT)], sg0).wait()
    wg = wct[pl.ds(0, L)]
    for e in range(CT):
        wb = lanebcast(wg, e)
        for f in range(E // L):
            rows0[e, pl.ds(f * L, L)] = rows0[e, pl.ds(f * L, L)] * wb
    pltpu.sync_copy(rows0.at[pl.ds(0, CT)], acc_sh.at[dstct], add=True)

    def sc_start(ci, srcc, sem):
        pltpu.async_copy(src_hbm.at[pl.ds(ebase + ci * CH, CH)], srcc, sem)

    def sc_wait(ci, srcc, sem):
        pltpu.make_async_copy(
            src_hbm.at[pl.ds(ebase + ci * CH, CH)], srcc, sem).wait()

    def g_start(srcc, rows, sem):
        pltpu.async_copy(sup_hbm.at[srcc], rows, sem)

    def g_wait(srcc, rows, sem):
        pltpu.make_async_copy(sup_hbm.at[srcc], rows, sem).wait()

    def d_start(ci, dstc, sem):
        pltpu.async_copy(dst_hbm.at[pl.ds(ebase + ci * CH, CH)], dstc, sem)

    def d_wait(ci, dstc, sem):
        pltpu.make_async_copy(
            dst_hbm.at[pl.ds(ebase + ci * CH, CH)], dstc, sem).wait()

    def w_start(ci, wc, sem):
        pltpu.async_copy(w_hbm.at[pl.ds(ebase + ci * CH, CH)], wc, sem)

    def w_wait(ci, wc, sem):
        pltpu.make_async_copy(
            w_hbm.at[pl.ds(ebase + ci * CH, CH)], wc, sem).wait()

    def s_start(rows, dstc, sem):
        pltpu.async_copy(rows, acc_sh.at[dstc], sem, add=True)

    def s_wait(rows, dstc, sem):
        pltpu.make_async_copy(rows, acc_sh.at[dstc], sem).wait()

    def scale(rows, wc):
        @pl.loop(0, CH // L)
        def _grp(g):
            wg = wc[pl.ds(g * L, L)]
            for e in range(L):
                row = g * L + e
                wb = lanebcast(wg, e)
                for f in range(E // L):
                    rows[row, pl.ds(f * L, L)] = (
                        rows[row, pl.ds(f * L, L)] * wb)

    sc_start(0, srcc0, ssc0)
    d_start(0, dstc0, sd0)
    w_start(0, wc0, sw0)
    sc_start(1, srcc1, ssc1)
    d_start(1, dstc1, sd1)
    w_start(1, wc1, sw1)
    sc_wait(0, srcc0, ssc0)
    g_start(srcc0, rows0, sg0)
    g_wait(srcc0, rows0, sg0)
    sc_start(2, srcc0, ssc0)
    sc_wait(1, srcc1, ssc1)
    g_start(srcc1, rows1, sg1)
    w_wait(0, wc0, sw0)
    scale(rows0, wc0)
    d_wait(0, dstc0, sd0)
    s_start(rows0, dstc0, ss0)

    NPAIR = (NCH - 2) // 2

    @pl.loop(0, NPAIR)
    def _pair(i):
        ci = 2 * i + 1
        g_wait(srcc1, rows1, sg1)
        sc_start(ci + 2, srcc1, ssc1)
        s_wait(rows0, dstc0, ss0)
        d_start(ci + 1, dstc0, sd0)
        w_start(ci + 1, wc0, sw0)
        sc_wait(ci + 1, srcc0, ssc0)
        g_start(srcc0, rows0, sg0)
        w_wait(ci, wc1, sw1)
        scale(rows1, wc1)
        d_wait(ci, dstc1, sd1)
        s_start(rows1, dstc1, ss1)
        g_wait(srcc0, rows0, sg0)

        @pl.when(i < NPAIR - 1)
        def _():
            sc_start(ci + 3, srcc0, ssc0)

        s_wait(rows1, dstc1, ss1)
        d_start(ci + 2, dstc1, sd1)
        w_start(ci + 2, wc1, sw1)
        sc_wait(ci + 2, srcc1, ssc1)
        g_start(srcc1, rows1, sg1)
        w_wait(ci + 1, wc0, sw0)
        scale(rows0, wc0)
        d_wait(ci + 1, dstc0, sd0)
        s_start(rows0, dstc0, ss0)

    g_wait(srcc1, rows1, sg1)
    s_wait(rows0, dstc0, ss0)
    w_wait(NCH - 1, wc1, sw1)
    scale(rows1, wc1)
    d_wait(NCH - 1, dstc1, sd1)
    s_start(rows1, dstc1, ss1)
    s_wait(rows1, dstc1, ss1)

    plsc.subcore_barrier()

    @pl.when(sid < NZT)
    def _():
        pltpu.sync_copy(acc_sh.at[pl.ds(sid * ZS, ZS)],
                        out_hbm.at[pl.ds(cid * NT + sid * ZS, ZS)])


@functools.partial(
    pl.kernel,
    out_type=[jax.ShapeDtypeStruct((GN, E), jnp.float32) for _ in range(4)],
    mesh=_MESH,
    scratch_types=[
        pltpu.VMEM((GPW,), jnp.int32),
        pltpu.VMEM((GC,), jnp.int32),
        pltpu.VMEM((GC, E), jnp.float32),
        pltpu.SemaphoreType.DMA,
    ],
)
def _gather_kernel(idx_hbm, t0, t1, t2, t3, o0, o1, o2, o3,
                   idx_v, idxc_v, rows_v, sem):
    cid = lax.axis_index("c")
    sid = lax.axis_index("s")
    wid = cid * NS + sid
    base = wid * GPW

    pltpu.sync_copy(idx_hbm.at[pl.ds(base, GPW)], idx_v)
    for g in range(GPW // L):
        gpos = jnp.full((L,), base + g * L, jnp.int32) + lax.iota(jnp.int32, L)
        v = idx_v[pl.ds(g * L, L)]
        off = jnp.where(gpos >= B,
                        jnp.full((L,), N_NUM, jnp.int32),
                        jnp.zeros((L,), jnp.int32))
        idx_v[pl.ds(g * L, L)] = v + off

    for tbl, out in ((t0, o0), (t1, o1), (t2, o2), (t3, o3)):
        for k in range(GPW // GC):
            for g in range(GC // L):
                idxc_v[pl.ds(g * L, L)] = idx_v[pl.ds(k * GC + g * L, L)]
            pltpu.async_copy(tbl.at[idxc_v], rows_v, sem).wait()
            pltpu.sync_copy(rows_v, out.at[pl.ds(base + k * GC, GC)])


LB = 1024
LSTEPS = B // LB
LBR = LB // E


def _loss_body(g0n, g0i, g0j, g1n, g1i, g1j, g2n, g2i, g2j, g3n, g3i, g3j,
               w0_ref, w1_ref, w2_ref, pi_ref, pj_ref, loss_ref, acc_ref):
    step = pl.program_id(0)
    pi = jnp.zeros((LB,), jnp.float32)
    pj = jnp.zeros((LB,), jnp.float32)
    sq = jnp.zeros((LB,), jnp.float32)
    for gn, gi, gj in ((g0n, g0i, g0j), (g1n, g1i, g1j),
                       (g2n, g2i, g2j), (g3n, g3i, g3j)):
        a = gn[...]
        bi = gi[...]
        bj = gj[...]
        pi = pi + jnp.sum(a * bi, axis=1)
        pj = pj + jnp.sum(a * bj, axis=1)
        sq = sq + jnp.sum(a * a + bi * bi + bj * bj, axis=1)
    pi_ref[...] = pi.reshape(LBR, E)
    pj_ref[...] = pj.reshape(LBR, E)

    z = pi - pj
    logsig = -(jnp.maximum(-z, 0.0) + jnp.log1p(jnp.exp(-jnp.abs(z))))

    @pl.when(step == 0)
    def _():
        acc_ref[0] = 0.0
        acc_ref[1] = 0.0

    acc_ref[0] = acc_ref[0] + jnp.sum(logsig)
    acc_ref[1] = acc_ref[1] + jnp.sum(sq)

    frob = (jnp.sqrt(jnp.sum(w0_ref[...] ** 2))
            + jnp.sqrt(jnp.sum(w1_ref[...] ** 2))
            + jnp.sqrt(jnp.sum(w2_ref[...] ** 2)))
    l2 = frob + acc_ref[1] / B
    loss = -acc_ref[0] / B + REG * l2
    loss_ref[...] = jnp.full((1, E), loss, jnp.float32)


def _loss(G0, G1, G2, G3, W0, W1, W2):
    g_specs = []
    g_args = []
    for G in (G0, G1, G2, G3):
        for p in range(3):
            g_specs.append(pl.BlockSpec(
                (LB, E), lambda i, p=p: (p * LSTEPS + i, 0)))
            g_args.append(G)
    w_specs = [pl.BlockSpec((E, E), lambda i: (0, 0)) for _ in range(3)]
    return pl.pallas_call(
        _loss_body,
        grid=(LSTEPS,),
        in_specs=g_specs + w_specs,
        out_specs=[pl.BlockSpec((LBR, E), lambda i: (i, 0)),
                   pl.BlockSpec((LBR, E), lambda i: (i, 0)),
                   pl.BlockSpec((1, E), lambda i: (0, 0))],
        out_shape=[jax.ShapeDtypeStruct((B // E, E), jnp.float32),
                   jax.ShapeDtypeStruct((B // E, E), jnp.float32),
                   jax.ShapeDtypeStruct((1, E), jnp.float32)],
        scratch_shapes=[pltpu.SMEM((2,), jnp.float32)],
    )(*g_args, W0, W1, W2)


def kernel(n, d_i, d_j, edge_index, edge_weight, E_weight, W0, W1, W2):
    src = edge_index[0].astype(jnp.int32)
    dst = edge_index[1].astype(jnp.int32)
    zeros_tbl = jnp.zeros((NT, E), jnp.float32)

    s0 = _mm_first(E_weight, W0)
    h = _edge_kernel(s0, src, dst, edge_weight, zeros_tbl)
    x1, s1 = _mm_sum(h[:NT], h[NT:], W1)
    h = _edge_kernel(s1, src, dst, edge_weight, zeros_tbl)
    x2, s2 = _mm_sum(h[:NT], h[NT:], W2)
    h = _edge_kernel(s2, src, dst, edge_weight, zeros_tbl)
    x3 = _add2(h[:NT], h[NT:])

    idx_cat = jnp.concatenate(
        [n.astype(jnp.int32), d_i.astype(jnp.int32), d_j.astype(jnp.int32)])
    G0, G1, G2, G3 = _gather_kernel(idx_cat, E_weight, x1, x2, x3)

    pre_i, pre_j, loss_buf = _loss(G0, G1, G2, G3, W0, W1, W2)
    return pre_i.reshape(B), pre_j.reshape(B), loss_buf[0, :1]


# speedup vs baseline: 9.7490x; 1.0673x over previous
import functools

import jax
import jax.numpy as jnp
from jax import lax
from jax.experimental import pallas as pl
from jax.experimental.pallas import tpu as pltpu
from jax.experimental.pallas import tpu_sc as plsc

N_NUM = 8000
D_NUM = 2000
NT = N_NUM + D_NUM
E = 128
NE = 320000
B = 4096
REG = 1e-4

NC, NS, L = 2, 16, 16
NW = NC * NS
EPW = NE // NW
CH = 128
NCH = EPW // CH
CT = EPW - NCH * CH
ZS = 1000
NZT = NT // ZS

GN = 3 * B
GPW = GN // NW
GC = 128

_MESH = plsc.VectorSubcoreMesh(
    core_axis_name="c", subcore_axis_name="s", num_cores=NC, num_subcores=NS)

MM_BLK = 1000
MM_GRID = NT // MM_BLK


def _mm_first_body(x_ref, w_ref, s_ref):
    s_ref[...] = lax.dot_general(
        x_ref[...], w_ref[...], (((1,), (1,)), ((), ())),
        preferred_element_type=jnp.float32)


def _mm_first(x, w):
    return pl.pallas_call(
        _mm_first_body,
        grid=(MM_GRID,),
        in_specs=[pl.BlockSpec((MM_BLK, E), lambda i: (i, 0)),
                  pl.BlockSpec((E, E), lambda i: (0, 0))],
        out_specs=pl.BlockSpec((MM_BLK, E), lambda i: (i, 0)),
        out_shape=jax.ShapeDtypeStruct((NT, E), jnp.float32),
    )(x, w)


def _mm_sum_body(a_ref, b_ref, w_ref, x_ref, s_ref):
    x = a_ref[...] + b_ref[...]
    x_ref[...] = x
    s_ref[...] = lax.dot_general(
        x, w_ref[...], (((1,), (1,)), ((), ())),
        preferred_element_type=jnp.float32)


def _mm_sum(a, b, w):
    return pl.pallas_call(
        _mm_sum_body,
        grid=(MM_GRID,),
        in_specs=[pl.BlockSpec((MM_BLK, E), lambda i: (i, 0)),
                  pl.BlockSpec((MM_BLK, E), lambda i: (i, 0)),
                  pl.BlockSpec((E, E), lambda i: (0, 0))],
        out_specs=[pl.BlockSpec((MM_BLK, E), lambda i: (i, 0)),
                   pl.BlockSpec((MM_BLK, E), lambda i: (i, 0))],
        out_shape=[jax.ShapeDtypeStruct((NT, E), jnp.float32),
                   jax.ShapeDtypeStruct((NT, E), jnp.float32)],
    )(a, b, w)


def _add_body(a_ref, b_ref, o_ref):
    o_ref[...] = a_ref[...] + b_ref[...]


def _add2(a, b):
    return pl.pallas_call(
        _add_body,
        grid=(MM_GRID,),
        in_specs=[pl.BlockSpec((MM_BLK, E), lambda i: (i, 0)),
                  pl.BlockSpec((MM_BLK, E), lambda i: (i, 0))],
        out_specs=pl.BlockSpec((MM_BLK, E), lambda i: (i, 0)),
        out_shape=jax.ShapeDtypeStruct((NT, E), jnp.float32),
    )(a, b)


@functools.partial(
    pl.kernel,
    out_type=jax.ShapeDtypeStruct((NC * NT, E), jnp.float32),
    mesh=_MESH,
    scratch_types=(
        [pltpu.VMEM((CH,), jnp.int32) for _ in range(3)]
        + [pltpu.VMEM((CH,), jnp.int32) for _ in range(3)]
        + [pltpu.VMEM((CH,), jnp.float32) for _ in range(3)]
        + [pltpu.VMEM((CT,), jnp.int32),
           pltpu.VMEM((CT,), jnp.int32),
           pltpu.VMEM((CT,), jnp.float32)]
        + [pltpu.VMEM((CH, E), jnp.float32) for _ in range(3)]
        + [pltpu.VMEM_SHARED((NT, E), jnp.float32)]
        + [pltpu.SemaphoreType.DMA for _ in range(15)]
    ),
)
def _edge_kernel(sup_hbm, src_hbm, dst_hbm, w_hbm, z_hbm, out_hbm,
                 srcc0, srcc1, srcc2, dstc0, dstc1, dstc2, wc0, wc1, wc2,
                 srcct, dstct, wct, rows0, rows1, rows2, acc_sh, *sems):
    sg, ss, sd, ssc, sw = (sems[0:3], sems[3:6], sems[6:9], sems[9:12],
                           sems[12:15])
    srcc = (srcc0, srcc1, srcc2)
    dstc = (dstc0, dstc1, dstc2)
    wc = (wc0, wc1, wc2)
    rows = (rows0, rows1, rows2)
    cid = lax.axis_index("c")
    sid = lax.axis_index("s")
    wid = cid * NS + sid
    ebase = wid * EPW

    @pl.when(sid < NZT)
    def _():
        pltpu.sync_copy(z_hbm.at[pl.ds(sid * ZS, ZS)],
                        acc_sh.at[pl.ds(sid * ZS, ZS)])

    plsc.subcore_barrier()

    def lanebcast(wg, e):
        return lax.gather(
            wg, jnp.full((L, 1), e, jnp.int32),
            lax.GatherDimensionNumbers(
                offset_dims=(), collapsed_slice_dims=(0,),
                start_index_map=(0,)),
            slice_sizes=(1,),
            mode=lax.GatherScatterMode.PROMISE_IN_BOUNDS)

    pltpu.sync_copy(src_hbm.at[pl.ds(ebase + NCH * CH, CT)], srcct)
    pltpu.sync_copy(dst_hbm.at[pl.ds(ebase + NCH * CH, CT)], dstct)
    pltpu.sync_copy(w_hbm.at[pl.ds(ebase + NCH * CH, CT)], wct)
    pltpu.async_copy(sup_hbm.at[srcct], rows0.at[pl.ds(0, CT)], sg[0]).wait()
    wg = wct[pl.ds(0, L)]
    for e in range(CT):
        wb = lanebcast(wg, e)
        for f in range(E // L):
            rows0[e, pl.ds(f * L, L)] = rows0[e, pl.ds(f * L, L)] * wb
    pltpu.sync_copy(rows0.at[pl.ds(0, CT)], acc_sh.at[dstct], add=True)

    def sc_start(ci, srcc, sem):
        pltpu.async_copy(src_hbm.at[pl.ds(ebase + ci * CH, CH)], srcc, sem)

    def sc_wait(ci, srcc, sem):
        pltpu.make_async_copy(
            src_hbm.at[pl.ds(ebase + ci * CH, CH)], srcc, sem).wait()

    def g_start(srcc, rows, sem):
        pltpu.async_copy(sup_hbm.at[srcc], rows, sem)

    def g_wait(srcc, rows, sem):
        pltpu.make_async_copy(sup_hbm.at[srcc], rows, sem).wait()

    def d_start(ci, dstc, sem):
        pltpu.async_copy(dst_hbm.at[pl.ds(ebase + ci * CH, CH)], dstc, sem)

    def d_wait(ci, dstc, sem):
        pltpu.make_async_copy(
            dst_hbm.at[pl.ds(ebase + ci * CH, CH)], dstc, sem).wait()

    def w_start(ci, wc, sem):
        pltpu.async_copy(w_hbm.at[pl.ds(ebase + ci * CH, CH)], wc, sem)

    def w_wait(ci, wc, sem):
        pltpu.make_async_copy(
            w_hbm.at[pl.ds(ebase + ci * CH, CH)], wc, sem).wait()

    def s_start(rows, dstc, sem):
        pltpu.async_copy(rows, acc_sh.at[dstc], sem, add=True)

    def s_wait(rows, dstc, sem):
        pltpu.make_async_copy(rows, acc_sh.at[dstc], sem).wait()

    def scale(rows, wc):
        @pl.loop(0, CH // L)
        def _grp(g):
            wg = wc[pl.ds(g * L, L)]
            for e in range(L):
                row = g * L + e
                wb = lanebcast(wg, e)
                for f in range(E // L):
                    rows[row, pl.ds(f * L, L)] = (
                        rows[row, pl.ds(f * L, L)] * wb)

    def slot(ci, k, kp, when_pre3=None, when_pre2=None):
        g_wait(srcc[k], rows[k], sg[k])

        def _pre3():
            sc_start(ci + 3, srcc[k], ssc[k])

        if when_pre3 is None:
            _pre3()
        else:
            pl.when(when_pre3)(_pre3)
        s_wait(rows[kp], dstc[kp], ss[kp])

        def _pre2():
            d_start(ci + 2, dstc[kp], sd[kp])
            w_start(ci + 2, wc[kp], sw[kp])
            sc_wait(ci + 2, srcc[kp], ssc[kp])
            g_start(srcc[kp], rows[kp], sg[kp])

        if when_pre2 is None:
            _pre2()
        else:
            pl.when(when_pre2)(_pre2)
        w_wait(ci, wc[k], sw[k])
        scale(rows[k], wc[k])
        d_wait(ci, dstc[k], sd[k])
        s_start(rows[k], dstc[k], ss[k])

    for j in range(3):
        sc_start(j, srcc[j], ssc[j])
        d_start(j, dstc[j], sd[j])
        w_start(j, wc[j], sw[j])
    sc_wait(0, srcc[0], ssc[0])
    g_start(srcc[0], rows[0], sg[0])
    sc_wait(1, srcc[1], ssc[1])
    g_start(srcc[1], rows[1], sg[1])

    g_wait(srcc[0], rows[0], sg[0])
    sc_start(3, srcc[0], ssc[0])
    sc_wait(2, srcc[2], ssc[2])
    g_start(srcc[2], rows[2], sg[2])
    w_wait(0, wc[0], sw[0])
    scale(rows[0], wc[0])
    d_wait(0, dstc[0], sd[0])
    s_start(rows[0], dstc[0], ss[0])

    slot(1, 1, 0)

    NLOOP = (NCH - 3) // 3

    @pl.loop(0, NLOOP)
    def _trio(i):
        slot(3 * i + 2, 2, 1)
        slot(3 * i + 3, 0, 2, when_pre3=(i < NLOOP - 1))
        slot(3 * i + 4, 1, 0, when_pre3=(i < NLOOP - 1),
             when_pre2=(i < NLOOP - 1))

    g_wait(srcc[2], rows[2], sg[2])
    s_wait(rows[1], dstc[1], ss[1])
    w_wait(NCH - 1, wc[2], sw[2])
    scale(rows[2], wc[2])
    d_wait(NCH - 1, dstc[2], sd[2])
    s_start(rows[2], dstc[2], ss[2])
    s_wait(rows[2], dstc[2], ss[2])

    plsc.subcore_barrier()

    @pl.when(sid < NZT)
    def _():
        pltpu.sync_copy(acc_sh.at[pl.ds(sid * ZS, ZS)],
                        out_hbm.at[pl.ds(cid * NT + sid * ZS, ZS)])


@functools.partial(
    pl.kernel,
    out_type=[jax.ShapeDtypeStruct((GN, E), jnp.float32) for _ in range(4)],
    mesh=_MESH,
    scratch_types=[
        pltpu.VMEM((GPW,), jnp.int32),
        pltpu.VMEM((GC,), jnp.int32),
        pltpu.VMEM((GC, E), jnp.float32),
        pltpu.SemaphoreType.DMA,
    ],
)
def _gather_kernel(idx_hbm, t0, t1, t2, t3, o0, o1, o2, o3,
                   idx_v, idxc_v, rows_v, sem):
    cid = lax.axis_index("c")
    sid = lax.axis_index("s")
    wid = cid * NS + sid
    base = wid * GPW

    pltpu.sync_copy(idx_hbm.at[pl.ds(base, GPW)], idx_v)
    for g in range(GPW // L):
        gpos = jnp.full((L,), base + g * L, jnp.int32) + lax.iota(jnp.int32, L)
        v = idx_v[pl.ds(g * L, L)]
        off = jnp.where(gpos >= B,
                        jnp.full((L,), N_NUM, jnp.int32),
                        jnp.zeros((L,), jnp.int32))
        idx_v[pl.ds(g * L, L)] = v + off

    for tbl, out in ((t0, o0), (t1, o1), (t2, o2), (t3, o3)):
        for k in range(GPW // GC):
            for g in range(GC // L):
                idxc_v[pl.ds(g * L, L)] = idx_v[pl.ds(k * GC + g * L, L)]
            pltpu.async_copy(tbl.at[idxc_v], rows_v, sem).wait()
            pltpu.sync_copy(rows_v, out.at[pl.ds(base + k * GC, GC)])


LB = 1024
LSTEPS = B // LB
LBR = LB // E


def _loss_body(g0n, g0i, g0j, g1n, g1i, g1j, g2n, g2i, g2j, g3n, g3i, g3j,
               w0_ref, w1_ref, w2_ref, pi_ref, pj_ref, loss_ref, acc_ref):
    step = pl.program_id(0)
    pi = jnp.zeros((LB,), jnp.float32)
    pj = jnp.zeros((LB,), jnp.float32)
    sq = jnp.zeros((LB,), jnp.float32)
    for gn, gi, gj in ((g0n, g0i, g0j), (g1n, g1i, g1j),
                       (g2n, g2i, g2j), (g3n, g3i, g3j)):
        a = gn[...]
        bi = gi[...]
        bj = gj[...]
        pi = pi + jnp.sum(a * bi, axis=1)
        pj = pj + jnp.sum(a * bj, axis=1)
        sq = sq + jnp.sum(a * a + bi * bi + bj * bj, axis=1)
    pi_ref[...] = pi.reshape(LBR, E)
    pj_ref[...] = pj.reshape(LBR, E)

    z = pi - pj
    logsig = -(jnp.maximum(-z, 0.0) + jnp.log1p(jnp.exp(-jnp.abs(z))))

    @pl.when(step == 0)
    def _():
        acc_ref[0] = 0.0
        acc_ref[1] = 0.0

    acc_ref[0] = acc_ref[0] + jnp.sum(logsig)
    acc_ref[1] = acc_ref[1] + jnp.sum(sq)

    frob = (jnp.sqrt(jnp.sum(w0_ref[...] ** 2))
            + jnp.sqrt(jnp.sum(w1_ref[...] ** 2))
            + jnp.sqrt(jnp.sum(w2_ref[...] ** 2)))
    l2 = frob + acc_ref[1] / B
    loss = -acc_ref[0] / B + REG * l2
    loss_ref[...] = jnp.full((1, E), loss, jnp.float32)


def _loss(G0, G1, G2, G3, W0, W1, W2):
    g_specs = []
    g_args = []
    for G in (G0, G1, G2, G3):
        for p in range(3):
            g_specs.append(pl.BlockSpec(
                (LB, E), lambda i, p=p: (p * LSTEPS + i, 0)))
            g_args.append(G)
    w_specs = [pl.BlockSpec((E, E), lambda i: (0, 0)) for _ in range(3)]
    return pl.pallas_call(
        _loss_body,
        grid=(LSTEPS,),
        in_specs=g_specs + w_specs,
        out_specs=[pl.BlockSpec((LBR, E), lambda i: (i, 0)),
                   pl.BlockSpec((LBR, E), lambda i: (i, 0)),
                   pl.BlockSpec((1, E), lambda i: (0, 0))],
        out_shape=[jax.ShapeDtypeStruct((B // E, E), jnp.float32),
                   jax.ShapeDtypeStruct((B // E, E), jnp.float32),
                   jax.ShapeDtypeStruct((1, E), jnp.float32)],
        scratch_shapes=[pltpu.SMEM((2,), jnp.float32)],
    )(*g_args, W0, W1, W2)


def kernel(n, d_i, d_j, edge_index, edge_weight, E_weight, W0, W1, W2):
    src = edge_index[0].astype(jnp.int32)
    dst = edge_index[1].astype(jnp.int32)
    zeros_tbl = jnp.zeros((NT, E), jnp.float32)

    s0 = _mm_first(E_weight, W0)
    h = _edge_kernel(s0, src, dst, edge_weight, zeros_tbl)
    x1, s1 = _mm_sum(h[:NT], h[NT:], W1)
    h = _edge_kernel(s1, src, dst, edge_weight, zeros_tbl)
    x2, s2 = _mm_sum(h[:NT], h[NT:], W2)
    h = _edge_kernel(s2, src, dst, edge_weight, zeros_tbl)
    x3 = _add2(h[:NT], h[NT:])

    idx_cat = jnp.concatenate(
        [n.astype(jnp.int32), d_i.astype(jnp.int32), d_j.astype(jnp.int32)])
    G0, G1, G2, G3 = _gather_kernel(idx_cat, E_weight, x1, x2, x3)

    pre_i, pre_j, loss_buf = _loss(G0, G1, G2, G3, W0, W1, W2)
    return pre_i.reshape(B), pre_j.reshape(B), loss_buf[0, :1]


# speedup vs baseline: 10.0298x; 1.0288x over previous
import functools

import jax
import jax.numpy as jnp
from jax import lax
from jax.experimental import pallas as pl
from jax.experimental.pallas import tpu as pltpu
from jax.experimental.pallas import tpu_sc as plsc

N_NUM = 8000
D_NUM = 2000
NT = N_NUM + D_NUM
E = 128
NE = 320000
B = 4096
REG = 1e-4

NC, NS, L = 2, 16, 16
NW = NC * NS
EPW = NE // NW
CH = 128
NCH = EPW // CH
CT = EPW - NCH * CH
ZS = 1000
NZT = NT // ZS

GN = 3 * B
GPW = GN // NW
GC = 128

_MESH = plsc.VectorSubcoreMesh(
    core_axis_name="c", subcore_axis_name="s", num_cores=NC, num_subcores=NS)

MM_BLK = 1000
MM_GRID = NT // MM_BLK


def _mm_first_body(x_ref, w_ref, s_ref):
    s_ref[...] = lax.dot_general(
        x_ref[...], w_ref[...], (((1,), (1,)), ((), ())),
        preferred_element_type=jnp.float32)


def _mm_first(x, w):
    return pl.pallas_call(
        _mm_first_body,
        grid=(MM_GRID,),
        in_specs=[pl.BlockSpec((MM_BLK, E), lambda i: (i, 0)),
                  pl.BlockSpec((E, E), lambda i: (0, 0))],
        out_specs=pl.BlockSpec((MM_BLK, E), lambda i: (i, 0)),
        out_shape=jax.ShapeDtypeStruct((NT, E), jnp.float32),
    )(x, w)


def _mm_sum_body(a_ref, b_ref, w_ref, x_ref, s_ref):
    x = a_ref[...] + b_ref[...]
    x_ref[...] = x
    s_ref[...] = lax.dot_general(
        x, w_ref[...], (((1,), (1,)), ((), ())),
        preferred_element_type=jnp.float32)


def _mm_sum(a, b, w):
    return pl.pallas_call(
        _mm_sum_body,
        grid=(MM_GRID,),
        in_specs=[pl.BlockSpec((MM_BLK, E), lambda i: (i, 0)),
                  pl.BlockSpec((MM_BLK, E), lambda i: (i, 0)),
                  pl.BlockSpec((E, E), lambda i: (0, 0))],
        out_specs=[pl.BlockSpec((MM_BLK, E), lambda i: (i, 0)),
                   pl.BlockSpec((MM_BLK, E), lambda i: (i, 0))],
        out_shape=[jax.ShapeDtypeStruct((NT, E), jnp.float32),
                   jax.ShapeDtypeStruct((NT, E), jnp.float32)],
    )(a, b, w)


@functools.partial(
    pl.kernel,
    out_type=jax.ShapeDtypeStruct((NC * NT, E), jnp.float32),
    mesh=_MESH,
    scratch_types=(
        [pltpu.VMEM((CH,), jnp.int32) for _ in range(3)]
        + [pltpu.VMEM((CH,), jnp.int32) for _ in range(3)]
        + [pltpu.VMEM((CH,), jnp.float32) for _ in range(3)]
        + [pltpu.VMEM((CT,), jnp.int32),
           pltpu.VMEM((CT,), jnp.int32),
           pltpu.VMEM((CT,), jnp.float32)]
        + [pltpu.VMEM((CH, E), jnp.float32) for _ in range(3)]
        + [pltpu.VMEM_SHARED((NT, E), jnp.float32)]
        + [pltpu.SemaphoreType.DMA for _ in range(15)]
    ),
)
def _edge_kernel(sup_hbm, src_hbm, dst_hbm, w_hbm, z_hbm, out_hbm,
                 srcc0, srcc1, srcc2, dstc0, dstc1, dstc2, wc0, wc1, wc2,
                 srcct, dstct, wct, rows0, rows1, rows2, acc_sh, *sems):
    sg, ss, sd, ssc, sw = (sems[0:3], sems[3:6], sems[6:9], sems[9:12],
                           sems[12:15])
    srcc = (srcc0, srcc1, srcc2)
    dstc = (dstc0, dstc1, dstc2)
    wc = (wc0, wc1, wc2)
    rows = (rows0, rows1, rows2)
    cid = lax.axis_index("c")
    sid = lax.axis_index("s")
    wid = cid * NS + sid
    ebase = wid * EPW

    def lanebcast(wg, e):
        return lax.gather(
            wg, jnp.full((L, 1), e, jnp.int32),
            lax.GatherDimensionNumbers(
                offset_dims=(), collapsed_slice_dims=(0,),
                start_index_map=(0,)),
            slice_sizes=(1,),
            mode=lax.GatherScatterMode.PROMISE_IN_BOUNDS)

    def sc_start(ci, srcc, sem):
        pltpu.async_copy(src_hbm.at[pl.ds(ebase + ci * CH, CH)], srcc, sem)

    def sc_wait(ci, srcc, sem):
        pltpu.make_async_copy(
            src_hbm.at[pl.ds(ebase + ci * CH, CH)], srcc, sem).wait()

    def g_start(srcc, rows, sem):
        pltpu.async_copy(sup_hbm.at[srcc], rows, sem)

    def g_wait(srcc, rows, sem):
        pltpu.make_async_copy(sup_hbm.at[srcc], rows, sem).wait()

    def d_start(ci, dstc, sem):
        pltpu.async_copy(dst_hbm.at[pl.ds(ebase + ci * CH, CH)], dstc, sem)

    def d_wait(ci, dstc, sem):
        pltpu.make_async_copy(
            dst_hbm.at[pl.ds(ebase + ci * CH, CH)], dstc, sem).wait()

    def w_start(ci, wc, sem):
        pltpu.async_copy(w_hbm.at[pl.ds(ebase + ci * CH, CH)], wc, sem)

    def w_wait(ci, wc, sem):
        pltpu.make_async_copy(
            w_hbm.at[pl.ds(ebase + ci * CH, CH)], wc, sem).wait()

    def s_start(rows, dstc, sem):
        pltpu.async_copy(rows, acc_sh.at[dstc], sem, add=True)

    def s_wait(rows, dstc, sem):
        pltpu.make_async_copy(rows, acc_sh.at[dstc], sem).wait()

    def scale(rows, wc):
        @pl.loop(0, CH // L)
        def _grp(g):
            wg = wc[pl.ds(g * L, L)]
            for e in range(L):
                row = g * L + e
                wb = lanebcast(wg, e)
                for f in range(E // L):
                    rows[row, pl.ds(f * L, L)] = (
                        rows[row, pl.ds(f * L, L)] * wb)

    def slot(ci, k, kp, when_pre3=None, when_pre2=None):
        g_wait(srcc[k], rows[k], sg[k])

        def _pre3():
            sc_start(ci + 3, srcc[k], ssc[k])

        if when_pre3 is None:
            _pre3()
        else:
            pl.when(when_pre3)(_pre3)
        s_wait(rows[kp], dstc[kp], ss[kp])

        def _pre2():
            d_start(ci + 2, dstc[kp], sd[kp])
            w_start(ci + 2, wc[kp], sw[kp])
            sc_wait(ci + 2, srcc[kp], ssc[kp])
            g_start(srcc[kp], rows[kp], sg[kp])

        if when_pre2 is None:
            _pre2()
        else:
            pl.when(when_pre2)(_pre2)
        w_wait(ci, wc[k], sw[k])
        scale(rows[k], wc[k])
        d_wait(ci, dstc[k], sd[k])
        s_start(rows[k], dstc[k], ss[k])

    for j in range(3):
        sc_start(j, srcc[j], ssc[j])
        d_start(j, dstc[j], sd[j])
        w_start(j, wc[j], sw[j])
    pltpu.sync_copy(src_hbm.at[pl.ds(ebase + NCH * CH, CT)], srcct)
    pltpu.sync_copy(dst_hbm.at[pl.ds(ebase + NCH * CH, CT)], dstct)
    pltpu.sync_copy(w_hbm.at[pl.ds(ebase + NCH * CH, CT)], wct)

    @pl.when(sid < NZT)
    def _():
        pltpu.sync_copy(z_hbm.at[pl.ds(sid * ZS, ZS)],
                        acc_sh.at[pl.ds(sid * ZS, ZS)])

    sc_wait(0, srcc[0], ssc[0])
    g_start(srcc[0], rows[0], sg[0])
    sc_wait(1, srcc[1], ssc[1])
    g_start(srcc[1], rows[1], sg[1])

    plsc.subcore_barrier()

    pltpu.async_copy(sup_hbm.at[srcct], rows2.at[pl.ds(0, CT)], ss[0]).wait()
    wgt = wct[pl.ds(0, L)]
    for e in range(CT):
        wb = lanebcast(wgt, e)
        for f in range(E // L):
            rows2[e, pl.ds(f * L, L)] = rows2[e, pl.ds(f * L, L)] * wb
    pltpu.sync_copy(rows2.at[pl.ds(0, CT)], acc_sh.at[dstct], add=True)

    g_wait(srcc[0], rows[0], sg[0])
    sc_start(3, srcc[0], ssc[0])
    sc_wait(2, srcc[2], ssc[2])
    g_start(srcc[2], rows[2], sg[2])
    w_wait(0, wc[0], sw[0])
    scale(rows[0], wc[0])
    d_wait(0, dstc[0], sd[0])
    s_start(rows[0], dstc[0], ss[0])

    slot(1, 1, 0)

    NLOOP = (NCH - 3) // 3

    @pl.loop(0, NLOOP)
    def _trio(i):
        slot(3 * i + 2, 2, 1)
        slot(3 * i + 3, 0, 2, when_pre3=(i < NLOOP - 1))
        slot(3 * i + 4, 1, 0, when_pre3=(i < NLOOP - 1),
             when_pre2=(i < NLOOP - 1))

    g_wait(srcc[2], rows[2], sg[2])
    s_wait(rows[1], dstc[1], ss[1])
    w_wait(NCH - 1, wc[2], sw[2])
    scale(rows[2], wc[2])
    d_wait(NCH - 1, dstc[2], sd[2])
    s_start(rows[2], dstc[2], ss[2])
    s_wait(rows[2], dstc[2], ss[2])

    plsc.subcore_barrier()

    @pl.when(sid < NZT)
    def _():
        pltpu.sync_copy(acc_sh.at[pl.ds(sid * ZS, ZS)],
                        out_hbm.at[pl.ds(cid * NT + sid * ZS, ZS)])


@functools.partial(
    pl.kernel,
    out_type=[jax.ShapeDtypeStruct((GN, E), jnp.float32) for _ in range(4)],
    mesh=_MESH,
    scratch_types=[
        pltpu.VMEM((GPW,), jnp.int32),
        pltpu.VMEM((GC,), jnp.int32),
        pltpu.VMEM((GC,), jnp.int32),
        pltpu.VMEM((GC, E), jnp.float32),
        pltpu.VMEM((GC, E), jnp.float32),
        pltpu.SemaphoreType.DMA,
    ],
)
def _gather_kernel(idx_hbm, t0, t1, t2, h_hbm, o0, o1, o2, o3,
                   idx_v, idxc_v, idxc2_v, rows_v, rows2_v, sem):
    cid = lax.axis_index("c")
    sid = lax.axis_index("s")
    wid = cid * NS + sid
    base = wid * GPW

    pltpu.sync_copy(idx_hbm.at[pl.ds(base, GPW)], idx_v)
    for g in range(GPW // L):
        gpos = jnp.full((L,), base + g * L, jnp.int32) + lax.iota(jnp.int32, L)
        v = idx_v[pl.ds(g * L, L)]
        off = jnp.where(gpos >= B,
                        jnp.full((L,), N_NUM, jnp.int32),
                        jnp.zeros((L,), jnp.int32))
        idx_v[pl.ds(g * L, L)] = v + off

    for tbl, out in ((t0, o0), (t1, o1), (t2, o2)):
        for k in range(GPW // GC):
            for g in range(GC // L):
                idxc_v[pl.ds(g * L, L)] = idx_v[pl.ds(k * GC + g * L, L)]
            pltpu.async_copy(tbl.at[idxc_v], rows_v, sem).wait()
            pltpu.sync_copy(rows_v, out.at[pl.ds(base + k * GC, GC)])

    for k in range(GPW // GC):
        for g in range(GC // L):
            v = idx_v[pl.ds(k * GC + g * L, L)]
            idxc_v[pl.ds(g * L, L)] = v
            idxc2_v[pl.ds(g * L, L)] = v + jnp.full((L,), NT, jnp.int32)
        pltpu.async_copy(h_hbm.at[idxc_v], rows_v, sem).wait()
        pltpu.async_copy(h_hbm.at[idxc2_v], rows2_v, sem).wait()

        @pl.loop(0, GC)
        def _addrow(r):
            for f in range(E // L):
                rows_v[r, pl.ds(f * L, L)] = (
                    rows_v[r, pl.ds(f * L, L)]
                    + rows2_v[r, pl.ds(f * L, L)])

        pltpu.sync_copy(rows_v, o3.at[pl.ds(base + k * GC, GC)])


LB = 1024
LSTEPS = B // LB
LBR = LB // E


def _loss_body(g0n, g0i, g0j, g1n, g1i, g1j, g2n, g2i, g2j, g3n, g3i, g3j,
               w0_ref, w1_ref, w2_ref, pi_ref, pj_ref, loss_ref, acc_ref):
    step = pl.program_id(0)
    pi = jnp.zeros((LB,), jnp.float32)
    pj = jnp.zeros((LB,), jnp.float32)
    sq = jnp.zeros((LB,), jnp.float32)
    for gn, gi, gj in ((g0n, g0i, g0j), (g1n, g1i, g1j),
                       (g2n, g2i, g2j), (g3n, g3i, g3j)):
        a = gn[...]
        bi = gi[...]
        bj = gj[...]
        pi = pi + jnp.sum(a * bi, axis=1)
        pj = pj + jnp.sum(a * bj, axis=1)
        sq = sq + jnp.sum(a * a + bi * bi + bj * bj, axis=1)
    pi_ref[...] = pi.reshape(LBR, E)
    pj_ref[...] = pj.reshape(LBR, E)

    z = pi - pj
    logsig = -(jnp.maximum(-z, 0.0) + jnp.log1p(jnp.exp(-jnp.abs(z))))

    @pl.when(step == 0)
    def _():
        acc_ref[0] = 0.0
        acc_ref[1] = 0.0

    acc_ref[0] = acc_ref[0] + jnp.sum(logsig)
    acc_ref[1] = acc_ref[1] + jnp.sum(sq)

    frob = (jnp.sqrt(jnp.sum(w0_ref[...] ** 2))
            + jnp.sqrt(jnp.sum(w1_ref[...] ** 2))
            + jnp.sqrt(jnp.sum(w2_ref[...] ** 2)))
    l2 = frob + acc_ref[1] / B
    loss = -acc_ref[0] / B + REG * l2
    loss_ref[...] = jnp.full((1, E), loss, jnp.float32)


def _loss(G0, G1, G2, G3, W0, W1, W2):
    g_specs = []
    g_args = []
    for G in (G0, G1, G2, G3):
        for p in range(3):
            g_specs.append(pl.BlockSpec(
                (LB, E), lambda i, p=p: (p * LSTEPS + i, 0)))
            g_args.append(G)
    w_specs = [pl.BlockSpec((E, E), lambda i: (0, 0)) for _ in range(3)]
    return pl.pallas_call(
        _loss_body,
        grid=(LSTEPS,),
        in_specs=g_specs + w_specs,
        out_specs=[pl.BlockSpec((LBR, E), lambda i: (i, 0)),
                   pl.BlockSpec((LBR, E), lambda i: (i, 0)),
                   pl.BlockSpec((1, E), lambda i: (0, 0))],
        out_shape=[jax.ShapeDtypeStruct((B // E, E), jnp.float32),
                   jax.ShapeDtypeStruct((B // E, E), jnp.float32),
                   jax.ShapeDtypeStruct((1, E), jnp.float32)],
        scratch_shapes=[pltpu.SMEM((2,), jnp.float32)],
    )(*g_args, W0, W1, W2)


def kernel(n, d_i, d_j, edge_index, edge_weight, E_weight, W0, W1, W2):
    src = edge_index[0].astype(jnp.int32)
    dst = edge_index[1].astype(jnp.int32)
    zeros_tbl = jnp.zeros((NT, E), jnp.float32)

    s0 = _mm_first(E_weight, W0)
    h = _edge_kernel(s0, src, dst, edge_weight, zeros_tbl)
    x1, s1 = _mm_sum(h[:NT], h[NT:], W1)
    h = _edge_kernel(s1, src, dst, edge_weight, zeros_tbl)
    x2, s2 = _mm_sum(h[:NT], h[NT:], W2)
    h = _edge_kernel(s2, src, dst, edge_weight, zeros_tbl)

    idx_cat = jnp.concatenate(
        [n.astype(jnp.int32), d_i.astype(jnp.int32), d_j.astype(jnp.int32)])
    G0, G1, G2, G3 = _gather_kernel(idx_cat, E_weight, x1, x2, h)

    pre_i, pre_j, loss_buf = _loss(G0, G1, G2, G3, W0, W1, W2)
    return pre_i.reshape(B), pre_j.reshape(B), loss_buf[0, :1]


# speedup vs baseline: 10.2532x; 1.0223x over previous
import functools

import jax
import jax.numpy as jnp
from jax import lax
from jax.experimental import pallas as pl
from jax.experimental.pallas import tpu as pltpu
from jax.experimental.pallas import tpu_sc as plsc

N_NUM = 8000
D_NUM = 2000
NT = N_NUM + D_NUM
E = 128
NE = 320000
B = 4096
REG = 1e-4

NC, NS, L = 2, 16, 16
NW = NC * NS
EPW = NE // NW
CH = 128
NCH = EPW // CH
CT = EPW - NCH * CH
ZS = 1000
NZT = NT // ZS

GN = 3 * B
GPW = GN // NW
GC = 192
NGC = GPW // GC

_MESH = plsc.VectorSubcoreMesh(
    core_axis_name="c", subcore_axis_name="s", num_cores=NC, num_subcores=NS)

MM_BLK = 1000
MM_GRID = NT // MM_BLK


def _mm_first_body(x_ref, w_ref, s_ref):
    s_ref[...] = lax.dot_general(
        x_ref[...], w_ref[...], (((1,), (1,)), ((), ())),
        preferred_element_type=jnp.float32)


def _mm_first(x, w):
    return pl.pallas_call(
        _mm_first_body,
        grid=(MM_GRID,),
        in_specs=[pl.BlockSpec((MM_BLK, E), lambda i: (i, 0)),
                  pl.BlockSpec((E, E), lambda i: (0, 0))],
        out_specs=pl.BlockSpec((MM_BLK, E), lambda i: (i, 0)),
        out_shape=jax.ShapeDtypeStruct((NT, E), jnp.float32),
    )(x, w)


def _mm_sum_body(a_ref, b_ref, w_ref, x_ref, s_ref):
    x = a_ref[...] + b_ref[...]
    x_ref[...] = x
    s_ref[...] = lax.dot_general(
        x, w_ref[...], (((1,), (1,)), ((), ())),
        preferred_element_type=jnp.float32)


def _mm_sum(a, b, w):
    return pl.pallas_call(
        _mm_sum_body,
        grid=(MM_GRID,),
        in_specs=[pl.BlockSpec((MM_BLK, E), lambda i: (i, 0)),
                  pl.BlockSpec((MM_BLK, E), lambda i: (i, 0)),
                  pl.BlockSpec((E, E), lambda i: (0, 0))],
        out_specs=[pl.BlockSpec((MM_BLK, E), lambda i: (i, 0)),
                   pl.BlockSpec((MM_BLK, E), lambda i: (i, 0))],
        out_shape=[jax.ShapeDtypeStruct((NT, E), jnp.float32),
                   jax.ShapeDtypeStruct((NT, E), jnp.float32)],
    )(a, b, w)


@functools.partial(
    pl.kernel,
    out_type=jax.ShapeDtypeStruct((NC * NT, E), jnp.float32),
    mesh=_MESH,
    scratch_types=(
        [pltpu.VMEM((CH,), jnp.int32) for _ in range(3)]
        + [pltpu.VMEM((CH,), jnp.int32) for _ in range(3)]
        + [pltpu.VMEM((CH,), jnp.float32) for _ in range(3)]
        + [pltpu.VMEM((CT,), jnp.int32),
           pltpu.VMEM((CT,), jnp.int32),
           pltpu.VMEM((CT,), jnp.float32)]
        + [pltpu.VMEM((CH, E), jnp.float32) for _ in range(3)]
        + [pltpu.VMEM_SHARED((NT, E), jnp.float32)]
        + [pltpu.SemaphoreType.DMA for _ in range(15)]
    ),
)
def _edge_kernel(sup_hbm, src_hbm, dst_hbm, w_hbm, z_hbm, out_hbm,
                 srcc0, srcc1, srcc2, dstc0, dstc1, dstc2, wc0, wc1, wc2,
                 srcct, dstct, wct, rows0, rows1, rows2, acc_sh, *sems):
    sg, ss, sd, ssc, sw = (sems[0:3], sems[3:6], sems[6:9], sems[9:12],
                           sems[12:15])
    srcc = (srcc0, srcc1, srcc2)
    dstc = (dstc0, dstc1, dstc2)
    wc = (wc0, wc1, wc2)
    rows = (rows0, rows1, rows2)
    cid = lax.axis_index("c")
    sid = lax.axis_index("s")
    wid = cid * NS + sid
    ebase = wid * EPW

    def lanebcast(wg, e):
        return lax.gather(
            wg, jnp.full((L, 1), e, jnp.int32),
            lax.GatherDimensionNumbers(
                offset_dims=(), collapsed_slice_dims=(0,),
                start_index_map=(0,)),
            slice_sizes=(1,),
            mode=lax.GatherScatterMode.PROMISE_IN_BOUNDS)

    def sc_start(ci, srcc, sem):
        pltpu.async_copy(src_hbm.at[pl.ds(ebase + ci * CH, CH)], srcc, sem)

    def sc_wait(ci, srcc, sem):
        pltpu.make_async_copy(
            src_hbm.at[pl.ds(ebase + ci * CH, CH)], srcc, sem).wait()

    def g_start(srcc, rows, sem):
        pltpu.async_copy(sup_hbm.at[srcc], rows, sem)

    def g_wait(srcc, rows, sem):
        pltpu.make_async_copy(sup_hbm.at[srcc], rows, sem).wait()

    def d_start(ci, dstc, sem):
        pltpu.async_copy(dst_hbm.at[pl.ds(ebase + ci * CH, CH)], dstc, sem)

    def d_wait(ci, dstc, sem):
        pltpu.make_async_copy(
            dst_hbm.at[pl.ds(ebase + ci * CH, CH)], dstc, sem).wait()

    def w_start(ci, wc, sem):
        pltpu.async_copy(w_hbm.at[pl.ds(ebase + ci * CH, CH)], wc, sem)

    def w_wait(ci, wc, sem):
        pltpu.make_async_copy(
            w_hbm.at[pl.ds(ebase + ci * CH, CH)], wc, sem).wait()

    def s_start(rows, dstc, sem):
        pltpu.async_copy(rows, acc_sh.at[dstc], sem, add=True)

    def s_wait(rows, dstc, sem):
        pltpu.make_async_copy(rows, acc_sh.at[dstc], sem).wait()

    def scale(rows, wc):
        @pl.loop(0, CH // L)
        def _grp(g):
            wg = wc[pl.ds(g * L, L)]
            for e in range(L):
                row = g * L + e
                wb = lanebcast(wg, e)
                for f in range(E // L):
                    rows[row, pl.ds(f * L, L)] = (
                        rows[row, pl.ds(f * L, L)] * wb)

    def slot(ci, k, kp, when_pre3=None, when_pre2=None):
        g_wait(srcc[k], rows[k], sg[k])

        def _pre3():
            sc_start(ci + 3, srcc[k], ssc[k])

        if when_pre3 is None:
            _pre3()
        else:
            pl.when(when_pre3)(_pre3)
        s_wait(rows[kp], dstc[kp], ss[kp])

        def _pre2():
            d_start(ci + 2, dstc[kp], sd[kp])
            w_start(ci + 2, wc[kp], sw[kp])
            sc_wait(ci + 2, srcc[kp], ssc[kp])
            g_start(srcc[kp], rows[kp], sg[kp])

        if when_pre2 is None:
            _pre2()
        else:
            pl.when(when_pre2)(_pre2)
        w_wait(ci, wc[k], sw[k])
        scale(rows[k], wc[k])
        d_wait(ci, dstc[k], sd[k])
        s_start(rows[k], dstc[k], ss[k])

    for j in range(3):
        sc_start(j, srcc[j], ssc[j])
        d_start(j, dstc[j], sd[j])
        w_start(j, wc[j], sw[j])
    pltpu.sync_copy(src_hbm.at[pl.ds(ebase + NCH * CH, CT)], srcct)
    pltpu.sync_copy(dst_hbm.at[pl.ds(ebase + NCH * CH, CT)], dstct)
    pltpu.sync_copy(w_hbm.at[pl.ds(ebase + NCH * CH, CT)], wct)

    @pl.when(sid < NZT)
    def _():
        pltpu.sync_copy(z_hbm.at[pl.ds(sid * ZS, ZS)],
                        acc_sh.at[pl.ds(sid * ZS, ZS)])

    sc_wait(0, srcc[0], ssc[0])
    g_start(srcc[0], rows[0], sg[0])
    sc_wait(1, srcc[1], ssc[1])
    g_start(srcc[1], rows[1], sg[1])

    plsc.subcore_barrier()

    pltpu.async_copy(sup_hbm.at[srcct], rows2.at[pl.ds(0, CT)], ss[0]).wait()
    wgt = wct[pl.ds(0, L)]
    for e in range(CT):
        wb = lanebcast(wgt, e)
        for f in range(E // L):
            rows2[e, pl.ds(f * L, L)] = rows2[e, pl.ds(f * L, L)] * wb
    pltpu.sync_copy(rows2.at[pl.ds(0, CT)], acc_sh.at[dstct], add=True)

    g_wait(srcc[0], rows[0], sg[0])
    sc_start(3, srcc[0], ssc[0])
    sc_wait(2, srcc[2], ssc[2])
    g_start(srcc[2], rows[2], sg[2])
    w_wait(0, wc[0], sw[0])
    scale(rows[0], wc[0])
    d_wait(0, dstc[0], sd[0])
    s_start(rows[0], dstc[0], ss[0])

    slot(1, 1, 0)

    NLOOP = (NCH - 3) // 3

    @pl.loop(0, NLOOP)
    def _trio(i):
        slot(3 * i + 2, 2, 1)
        slot(3 * i + 3, 0, 2, when_pre3=(i < NLOOP - 1))
        slot(3 * i + 4, 1, 0, when_pre3=(i < NLOOP - 1),
             when_pre2=(i < NLOOP - 1))

    g_wait(srcc[2], rows[2], sg[2])
    s_wait(rows[1], dstc[1], ss[1])
    w_wait(NCH - 1, wc[2], sw[2])
    scale(rows[2], wc[2])
    d_wait(NCH - 1, dstc[2], sd[2])
    s_start(rows[2], dstc[2], ss[2])
    s_wait(rows[2], dstc[2], ss[2])

    plsc.subcore_barrier()

    @pl.when(sid < NZT)
    def _():
        pltpu.sync_copy(acc_sh.at[pl.ds(sid * ZS, ZS)],
                        out_hbm.at[pl.ds(cid * NT + sid * ZS, ZS)])


@functools.partial(
    pl.kernel,
    out_type=[jax.ShapeDtypeStruct((GN, E), jnp.float32) for _ in range(4)],
    mesh=_MESH,
    scratch_types=(
        [pltpu.VMEM((GPW,), jnp.int32)]
        + [pltpu.VMEM((GC,), jnp.int32) for _ in range(2)]
        + [pltpu.VMEM((GC,), jnp.int32) for _ in range(2)]
        + [pltpu.VMEM((GC, E), jnp.float32) for _ in range(2)]
        + [pltpu.VMEM((GC, E), jnp.float32) for _ in range(2)]
        + [pltpu.SemaphoreType.DMA for _ in range(6)]
    ),
)
def _gather_kernel(idx_hbm, t0, t1, t2, h_hbm, o0, o1, o2, o3,
                   idx_v, idxc0, idxc1, idxh0, idxh1, rowsa0, rowsa1,
                   rowsh0, rowsh1, *sems):
    idxc = (idxc0, idxc1)
    idxh = (idxh0, idxh1)
    rows = (rowsa0, rowsa1)
    rowsh = (rowsh0, rowsh1)
    sg, sgh, sco = sems[0:2], sems[2:4], sems[4:6]
    cid = lax.axis_index("c")
    sid = lax.axis_index("s")
    wid = cid * NS + sid
    base = wid * GPW

    pltpu.sync_copy(idx_hbm.at[pl.ds(base, GPW)], idx_v)
    for g in range(GPW // L):
        gpos = jnp.full((L,), base + g * L, jnp.int32) + lax.iota(jnp.int32, L)
        v = idx_v[pl.ds(g * L, L)]
        off = jnp.where(gpos >= B,
                        jnp.full((L,), N_NUM, jnp.int32),
                        jnp.zeros((L,), jnp.int32))
        idx_v[pl.ds(g * L, L)] = v + off

    jobs = ([(tbl, out, k, False)
             for tbl, out in ((t0, o0), (t1, o1), (t2, o2))
             for k in range(NGC)]
            + [(h_hbm, o3, k, True) for k in range(NGC)])
    NJ = len(jobs)

    def stage_and_fire(j):
        tbl, _, k, is_h = jobs[j]
        b = j % 2
        for g in range(GC // L):
            v = idx_v[pl.ds(k * GC + g * L, L)]
            idxc[b][pl.ds(g * L, L)] = v
            if is_h:
                idxh[b][pl.ds(g * L, L)] = v + jnp.full((L,), NT, jnp.int32)
        pltpu.async_copy(tbl.at[idxc[b]], rows[b], sg[b])
        if is_h:
            pltpu.async_copy(tbl.at[idxh[b]], rowsh[b], sgh[b])

    def co_wait(j):
        _, out, k, _ = jobs[j]
        b = j % 2
        pltpu.make_async_copy(
            rows[b], out.at[pl.ds(base + k * GC, GC)], sco[b]).wait()

    stage_and_fire(0)
    for j in range(NJ):
        tbl, out, k, is_h = jobs[j]
        b = j % 2
        if j + 1 < NJ:
            if j >= 1:
                co_wait(j - 1)
            stage_and_fire(j + 1)
        pltpu.make_async_copy(tbl.at[idxc[b]], rows[b], sg[b]).wait()
        if is_h:
            pltpu.make_async_copy(tbl.at[idxh[b]], rowsh[b], sgh[b]).wait()

            @pl.loop(0, GC)
            def _addrow(r):
                for f in range(E // L):
                    rows[b][r, pl.ds(f * L, L)] = (
                        rows[b][r, pl.ds(f * L, L)]
                        + rowsh[b][r, pl.ds(f * L, L)])

        pltpu.async_copy(rows[b], out.at[pl.ds(base + k * GC, GC)], sco[b])
    co_wait(NJ - 2)
    co_wait(NJ - 1)


LB = 1024
LSTEPS = B // LB
LBR = LB // E


def _loss_body(g0n, g0i, g0j, g1n, g1i, g1j, g2n, g2i, g2j, g3n, g3i, g3j,
               w0_ref, w1_ref, w2_ref, pi_ref, pj_ref, loss_ref, acc_ref):
    step = pl.program_id(0)
    pi = jnp.zeros((LB,), jnp.float32)
    pj = jnp.zeros((LB,), jnp.float32)
    sq = jnp.zeros((LB,), jnp.float32)
    for gn, gi, gj in ((g0n, g0i, g0j), (g1n, g1i, g1j),
                       (g2n, g2i, g2j), (g3n, g3i, g3j)):
        a = gn[...]
        bi = gi[...]
        bj = gj[...]
        pi = pi + jnp.sum(a * bi, axis=1)
        pj = pj + jnp.sum(a * bj, axis=1)
        sq = sq + jnp.sum(a * a + bi * bi + bj * bj, axis=1)
    pi_ref[...] = pi.reshape(LBR, E)
    pj_ref[...] = pj.reshape(LBR, E)

    z = pi - pj
    logsig = -(jnp.maximum(-z, 0.0) + jnp.log1p(jnp.exp(-jnp.abs(z))))

    @pl.when(step == 0)
    def _():
        acc_ref[0] = 0.0
        acc_ref[1] = 0.0

    acc_ref[0] = acc_ref[0] + jnp.sum(logsig)
    acc_ref[1] = acc_ref[1] + jnp.sum(sq)

    frob = (jnp.sqrt(jnp.sum(w0_ref[...] ** 2))
            + jnp.sqrt(jnp.sum(w1_ref[...] ** 2))
            + jnp.sqrt(jnp.sum(w2_ref[...] ** 2)))
    l2 = frob + acc_ref[1] / B
    loss = -acc_ref[0] / B + REG * l2
    loss_ref[...] = jnp.full((1, E), loss, jnp.float32)


def _loss(G0, G1, G2, G3, W0, W1, W2):
    g_specs = []
    g_args = []
    for G in (G0, G1, G2, G3):
        for p in range(3):
            g_specs.append(pl.BlockSpec(
                (LB, E), lambda i, p=p: (p * LSTEPS + i, 0)))
            g_args.append(G)
    w_specs = [pl.BlockSpec((E, E), lambda i: (0, 0)) for _ in range(3)]
    return pl.pallas_call(
        _loss_body,
        grid=(LSTEPS,),
        in_specs=g_specs + w_specs,
        out_specs=[pl.BlockSpec((LBR, E), lambda i: (i, 0)),
                   pl.BlockSpec((LBR, E), lambda i: (i, 0)),
                   pl.BlockSpec((1, E), lambda i: (0, 0))],
        out_shape=[jax.ShapeDtypeStruct((B // E, E), jnp.float32),
                   jax.ShapeDtypeStruct((B // E, E), jnp.float32),
                   jax.ShapeDtypeStruct((1, E), jnp.float32)],
        scratch_shapes=[pltpu.SMEM((2,), jnp.float32)],
    )(*g_args, W0, W1, W2)


def kernel(n, d_i, d_j, edge_index, edge_weight, E_weight, W0, W1, W2):
    src = edge_index[0].astype(jnp.int32)
    dst = edge_index[1].astype(jnp.int32)
    zeros_tbl = jnp.zeros((NT, E), jnp.float32)

    s0 = _mm_first(E_weight, W0)
    h = _edge_kernel(s0, src, dst, edge_weight, zeros_tbl)
    x1, s1 = _mm_sum(h[:NT], h[NT:], W1)
    h = _edge_kernel(s1, src, dst, edge_weight, zeros_tbl)
    x2, s2 = _mm_sum(h[:NT], h[NT:], W2)
    h = _edge_kernel(s2, src, dst, edge_weight, zeros_tbl)

    idx_cat = jnp.concatenate(
        [n.astype(jnp.int32), d_i.astype(jnp.int32), d_j.astype(jnp.int32)])
    G0, G1, G2, G3 = _gather_kernel(idx_cat, E_weight, x1, x2, h)

    pre_i, pre_j, loss_buf = _loss(G0, G1, G2, G3, W0, W1, W2)
    return pre_i.reshape(B), pre_j.reshape(B), loss_buf[0, :1]
